# baseline (speedup 1.0000x reference)
import functools

import jax
import jax.numpy as jnp
from jax import lax
from jax.experimental import pallas as pl
from jax.experimental.pallas import tpu as pltpu
from jax.experimental.pallas import tpu_sc as plsc

N_TOK = 8192
NUM_IN = 256
TOTAL_OUT = 256
HALF_OUT = 128
E = 16
NTYPES = 128
TEBD = 64

TM = 256
CAP = 12288
NB = CAP // TM
TM_SHIFT = TM.bit_length() - 1

NC, NS = 2, 16
NW = NC * NS
TPT = N_TOK // NW
RPT = TPT // 128
NV = N_TOK // 16


def _wid():
    return lax.axis_index("s") * NC + lax.axis_index("c")


def _mesh():
    return plsc.VectorSubcoreMesh(core_axis_name="c", subcore_axis_name="s")


_SC_PARAMS = pltpu.CompilerParams(needs_layout_passes=False)


def _emap_body(te_ref, gw_ref, gb_ref, em_ref):
    logits = jnp.dot(te_ref[...], gw_ref[...],
                     preferred_element_type=jnp.float32) + gb_ref[...]
    em_ref[...] = jnp.argmax(logits, axis=1).astype(jnp.int32)[None, :]


def _expert_map(type_embeddings, gate_W, gate_b):
    return pl.pallas_call(
        _emap_body,
        out_shape=jax.ShapeDtypeStruct((1, NTYPES), jnp.int32),
    )(type_embeddings, gate_W, gate_b.reshape(1, E))


def _sort_body(em_hbm, at_hbm, x_hbm, pos_hbm, bexp_hbm, xs_hbm,
               em_v, at_v, ev_v, hist_v, ctr_v, pos2_v, bexp_v, xv_v,
               sem, xsem):
    wid = _wid()
    lane = lax.broadcasted_iota(jnp.int32, (E,), 0)
    ones = jnp.ones((E,), jnp.int32)

    xload = pltpu.async_copy(x_hbm.at[pl.ds(wid * TPT, TPT)], xv_v, xsem)
    pltpu.sync_copy(em_hbm.at[0], em_v)
    pltpu.sync_copy(at_hbm, at_v)

    for r in range(8):
        hist_v[pl.ds(r * 16, 16)] = jnp.zeros((E,), jnp.int32)

    def _acc8(i, carry):
        for r in range(8):
            tv = at_v[pl.ds((i * 8 + r) * 16, 16)]
            evv = plsc.load_gather(em_v, [tv])
            plsc.addupdate_scatter(hist_v, [r * 16 + evv], ones)
        return carry

    def _hsum():
        h = jnp.zeros((E,), jnp.int32)
        for r in range(8):
            h = h + hist_v[pl.ds(r * 16, 16)]
        return h

    lax.fori_loop(0, wid * (TPT // 128), _acc8, 0)
    bef = _hsum()
    base = wid * (TPT // 16)

    def _acc_own(i, carry):
        tv = at_v[pl.ds((base + i) * 16, 16)]
        evv = plsc.load_gather(em_v, [tv])
        ev_v[pl.ds(i * 16, 16)] = evv
        plsc.addupdate_scatter(hist_v, [(i & 7) * 16 + evv], ones)
        return carry
    lax.fori_loop(0, TPT // 16, _acc_own, 0)
    lax.fori_loop((base + TPT // 16) // 8, NV // 8, _acc8, 0)
    g = _hsum()

    padded = ((g + (TM - 1)) >> TM_SHIFT) << TM_SHIFT
    offs = plsc.cumsum(padded) - padded
    start = offs + bef

    @pl.when(wid == 0)
    def _():
        for k in range(NB // 16):
            bv = (lane + 16 * k) * TM
            be = jnp.zeros((E,), jnp.int32)
            for e in range(E):
                be = jnp.where((bv >= offs[e]) & (bv < offs[e] + padded[e]),
                               e, be)
            bexp_v[pl.ds(16 * k, 16)] = be
        pltpu.sync_copy(bexp_v, bexp_hbm)

    ctr_v[...] = start

    def _rank(i, carry):
        evv = ev_v[pl.ds(i * 16, 16)]
        base_ = plsc.load_gather(ctr_v, [evv])
        rk, _ = plsc.scan_count(evv)
        posv = jnp.minimum(base_ + rk - 1, CAP - 1)
        li = i * 16 + lane
        plsc.store_scatter(pos2_v, [li >> 7, li & 127], posv)
        plsc.addupdate_scatter(ctr_v, [evv], ones)
        return carry
    lax.fori_loop(0, TPT // 16, _rank, 0)

    pltpu.sync_copy(pos2_v, pos_hbm.at[pl.ds(wid * RPT, RPT)])
    xload.wait()
    cps = [pltpu.async_copy(xv_v.at[pl.ds(j * 128, 128)],
                            xs_hbm.at[pos2_v.at[j]], sem)
           for j in range(RPT)]
    for c in cps:
        c.wait()


def _sort_scatter(em, atom_types, x):
    f = functools.partial(
        pl.kernel,
        out_type=[jax.ShapeDtypeStruct((N_TOK // 128, 128), jnp.int32),
                  jax.ShapeDtypeStruct((NB,), jnp.int32),
                  jax.ShapeDtypeStruct((CAP, NUM_IN // 2), jnp.int32)],
        mesh=_mesh(),
        scratch_types=[
            pltpu.VMEM((NTYPES,), jnp.int32),
            pltpu.VMEM((N_TOK,), jnp.int32),
            pltpu.VMEM((TPT,), jnp.int32),
            pltpu.VMEM((8 * E,), jnp.int32),
            pltpu.VMEM((E,), jnp.int32),
            pltpu.VMEM((RPT, 128), jnp.int32),
            pltpu.VMEM((NB,), jnp.int32),
            pltpu.VMEM((TPT, NUM_IN // 2), jnp.int32),
            pltpu.SemaphoreType.DMA,
            pltpu.SemaphoreType.DMA,
        ],
        compiler_params=_SC_PARAMS,
    )(_sort_body)
    return f(em, atom_types, x)


BPS = 4
SM = TM * BPS
NG = CAP // SM


def _gemm_body(bexp_ref, xs_ref, w_ref, b_ref, y1_ref, y2_ref):
    gidx = pl.program_id(0)
    for k in range(BPS):
        e = bexp_ref[gidx * BPS + k]
        xw = xs_ref[pl.ds(k * TM, TM), :]
        xb = pltpu.bitcast(xw, jnp.bfloat16).reshape(TM, NUM_IN)
        w = w_ref[pl.ds(e, 1), :, :][0]
        y = jnp.tanh(
            jnp.dot(xb, w, preferred_element_type=jnp.float32)
            + b_ref[pl.ds(e, 1), :])
        y1_ref[pl.ds(k * TM, TM), :] = y[:, :HALF_OUT]
        y2_ref[pl.ds(k * TM, TM), :] = y[:, HALF_OUT:]


def _gemm(bexp, xs, expert_W_bf16, expert_b):
    grid_spec = pltpu.PrefetchScalarGridSpec(
        num_scalar_prefetch=1,
        grid=(NG,),
        in_specs=[
            pl.BlockSpec((SM, NUM_IN // 2), lambda b, s: (b, 0)),
            pl.BlockSpec((E, NUM_IN, TOTAL_OUT), lambda b, s: (0, 0, 0)),
            pl.BlockSpec((E, TOTAL_OUT), lambda b, s: (0, 0)),
        ],
        out_specs=[
            pl.BlockSpec((SM, HALF_OUT), lambda b, s: (b, 0)),
            pl.BlockSpec((SM, HALF_OUT), lambda b, s: (b, 0)),
        ],
    )
    return pl.pallas_call(
        _gemm_body,
        grid_spec=grid_spec,
        out_shape=[jax.ShapeDtypeStruct((CAP, HALF_OUT), jnp.float32),
                   jax.ShapeDtypeStruct((CAP, HALF_OUT), jnp.float32)],
    )(bexp, xs, expert_W_bf16, expert_b)


def _gath_body(pos_hbm, y1_hbm, y2_hbm, o1_hbm, o2_hbm,
               pv_v, y1_v, y2_v, sem, wsem):
    wid = _wid()
    pltpu.sync_copy(pos_hbm.at[pl.ds(wid * RPT, RPT)], pv_v)
    g1 = [pltpu.async_copy(y1_hbm.at[pv_v.at[j]],
                           y1_v.at[pl.ds(j * 128, 128)], sem)
          for j in range(RPT)]
    g2 = [pltpu.async_copy(y2_hbm.at[pv_v.at[j]],
                           y2_v.at[pl.ds(j * 128, 128)], sem)
          for j in range(RPT)]
    for c in g1:
        c.wait()
    w1 = pltpu.async_copy(y1_v, o1_hbm.at[pl.ds(wid * TPT, TPT)], wsem)
    for c in g2:
        c.wait()
    w2 = pltpu.async_copy(y2_v, o2_hbm.at[pl.ds(wid * TPT, TPT)], wsem)
    w1.wait()
    w2.wait()


def _gather_y(pos, y1, y2):
    f = functools.partial(
        pl.kernel,
        out_type=[jax.ShapeDtypeStruct((N_TOK, HALF_OUT), jnp.float32),
                  jax.ShapeDtypeStruct((N_TOK, HALF_OUT), jnp.float32)],
        mesh=_mesh(),
        scratch_types=[
            pltpu.VMEM((RPT, 128), jnp.int32),
            pltpu.VMEM((TPT, HALF_OUT), jnp.float32),
            pltpu.VMEM((TPT, HALF_OUT), jnp.float32),
            pltpu.SemaphoreType.DMA,
            pltpu.SemaphoreType.DMA,
        ],
        compiler_params=_SC_PARAMS,
    )(_gath_body)
    return f(pos, y1, y2)


def kernel(x, type_embeddings, atom_types, gate_W, gate_b, expert_W, expert_b):
    atom_types = atom_types.astype(jnp.int32)
    em = _expert_map(type_embeddings, gate_W, gate_b)
    xi = lax.bitcast_convert_type(
        x.astype(jnp.bfloat16).reshape(N_TOK, NUM_IN // 2, 2), jnp.int32)
    pos, bexp, xs = _sort_scatter(em, atom_types, xi)
    wb = expert_W.astype(jnp.bfloat16)
    wp = jnp.concatenate([wb[:, 0::2, :], wb[:, 1::2, :]], axis=1)
    y1, y2 = _gemm(bexp, xs, wp, expert_b)
    o1, o2 = _gather_y(pos, y1, y2)
    return (o1, o2)


# speedup vs baseline: 1.5289x; 1.5289x over previous
import functools

import jax
import jax.numpy as jnp
from jax import lax
from jax.experimental import pallas as pl
from jax.experimental.pallas import tpu as pltpu
from jax.experimental.pallas import tpu_sc as plsc

N_TOK = 8192
NUM_IN = 256
TOTAL_OUT = 256
HALF_OUT = 128
E = 16
NTYPES = 128
TEBD = 64

TM = 256
CAP = 12288
NB = CAP // TM
TM_SHIFT = TM.bit_length() - 1

NC, NS = 2, 16
NW = NC * NS
TPT = N_TOK // NW
RPT = TPT // 128
NV = N_TOK // 16


def _wid():
    return lax.axis_index("s") * NC + lax.axis_index("c")


def _mesh():
    return plsc.VectorSubcoreMesh(core_axis_name="c", subcore_axis_name="s")


_SC_PARAMS = pltpu.CompilerParams(needs_layout_passes=False)


def _emap_body(te_ref, gw_ref, gb_ref, em_ref):
    logits = jnp.dot(te_ref[...], gw_ref[...],
                     preferred_element_type=jnp.float32) + gb_ref[...]
    em_ref[...] = jnp.argmax(logits, axis=1).astype(jnp.int32)[None, :]


def _expert_map(type_embeddings, gate_W, gate_b):
    return pl.pallas_call(
        _emap_body,
        out_shape=jax.ShapeDtypeStruct((1, NTYPES), jnp.int32),
    )(type_embeddings, gate_W, gate_b.reshape(1, E))


def _sort_body(em_hbm, at_hbm, x_hbm, pos_hbm, bexp_hbm, xs_hbm,
               em_v, at_v, ev_v, hist_v, ctr_v, pos2_v, bexp_v, xv_v,
               sem, xsem):
    wid = _wid()
    lane = lax.broadcasted_iota(jnp.int32, (E,), 0)
    ones = jnp.ones((E,), jnp.int32)

    xload = pltpu.async_copy(x_hbm.at[pl.ds(wid * TPT, TPT)], xv_v, xsem)
    pltpu.sync_copy(em_hbm.at[0], em_v)
    pltpu.sync_copy(at_hbm, at_v)

    for r in range(8):
        hist_v[pl.ds(r * 16, 16)] = jnp.zeros((E,), jnp.int32)

    def _acc8(i, carry):
        for r in range(8):
            tv = at_v[pl.ds((i * 8 + r) * 16, 16)]
            evv = plsc.load_gather(em_v, [tv])
            plsc.addupdate_scatter(hist_v, [r * 16 + evv], ones)
        return carry

    def _hsum():
        h = jnp.zeros((E,), jnp.int32)
        for r in range(8):
            h = h + hist_v[pl.ds(r * 16, 16)]
        return h

    lax.fori_loop(0, wid * (TPT // 128), _acc8, 0)
    bef = _hsum()
    base = wid * (TPT // 16)

    def _acc_own(i, carry):
        tv = at_v[pl.ds((base + i) * 16, 16)]
        evv = plsc.load_gather(em_v, [tv])
        ev_v[pl.ds(i * 16, 16)] = evv
        plsc.addupdate_scatter(hist_v, [(i & 7) * 16 + evv], ones)
        return carry
    lax.fori_loop(0, TPT // 16, _acc_own, 0)
    lax.fori_loop((base + TPT // 16) // 8, NV // 8, _acc8, 0)
    g = _hsum()

    padded = ((g + (TM - 1)) >> TM_SHIFT) << TM_SHIFT
    offs = plsc.cumsum(padded) - padded
    start = offs + bef

    @pl.when(wid == 0)
    def _():
        for k in range(NB // 16):
            bv = (lane + 16 * k) * TM
            be = jnp.zeros((E,), jnp.int32)
            for e in range(E):
                be = jnp.where((bv >= offs[e]) & (bv < offs[e] + padded[e]),
                               e, be)
            bexp_v[pl.ds(16 * k, 16)] = be
        pltpu.sync_copy(bexp_v, bexp_hbm)

    ctr_v[...] = start

    def _rank(i, carry):
        evv = ev_v[pl.ds(i * 16, 16)]
        base_ = plsc.load_gather(ctr_v, [evv])
        rk, _ = plsc.scan_count(evv)
        posv = jnp.minimum(base_ + rk - 1, CAP - 1)
        li = i * 16 + lane
        plsc.store_scatter(pos2_v, [li >> 7, li & 127], posv)
        plsc.addupdate_scatter(ctr_v, [evv], ones)
        return carry
    lax.fori_loop(0, TPT // 16, _rank, 0)

    pltpu.sync_copy(pos2_v, pos_hbm.at[pl.ds(wid * RPT, RPT)])
    xload.wait()
    cps = [pltpu.async_copy(xv_v.at[pl.ds(j * 128, 128)],
                            xs_hbm.at[pos2_v.at[j]], sem)
           for j in range(RPT)]
    for c in cps:
        c.wait()


def _sort_scatter(em, atom_types, x):
    f = functools.partial(
        pl.kernel,
        out_type=[jax.ShapeDtypeStruct((N_TOK // 128, 128), jnp.int32),
                  jax.ShapeDtypeStruct((NB,), jnp.int32),
                  jax.ShapeDtypeStruct((CAP, NUM_IN // 2), jnp.int32)],
        mesh=_mesh(),
        scratch_types=[
            pltpu.VMEM((NTYPES,), jnp.int32),
            pltpu.VMEM((N_TOK,), jnp.int32),
            pltpu.VMEM((TPT,), jnp.int32),
            pltpu.VMEM((8 * E,), jnp.int32),
            pltpu.VMEM((E,), jnp.int32),
            pltpu.VMEM((RPT, 128), jnp.int32),
            pltpu.VMEM((NB,), jnp.int32),
            pltpu.VMEM((TPT, NUM_IN // 2), jnp.int32),
            pltpu.SemaphoreType.DMA,
            pltpu.SemaphoreType.DMA,
        ],
        compiler_params=_SC_PARAMS,
    )(_sort_body)
    return f(em, atom_types, x)


BPS = 4
SM = TM * BPS
NG = CAP // SM


def _gemm_body(bexp_ref, xs_ref, w_ref, b_ref, y1_ref, y2_ref):
    gidx = pl.program_id(0)
    for k in range(BPS):
        e = bexp_ref[gidx * BPS + k]
        xw = xs_ref[pl.ds(k * TM, TM), :]
        xb = pltpu.bitcast(xw, jnp.bfloat16).reshape(TM, NUM_IN)
        w = w_ref[pl.ds(e, 1), :, :][0]
        y = jnp.tanh(
            jnp.dot(xb, w, preferred_element_type=jnp.float32)
            + b_ref[pl.ds(e, 1), :])
        y1_ref[pl.ds(k * TM, TM), :] = y[:, :HALF_OUT]
        y2_ref[pl.ds(k * TM, TM), :] = y[:, HALF_OUT:]


def _gemm(bexp, xs, expert_W_bf16, expert_b):
    grid_spec = pltpu.PrefetchScalarGridSpec(
        num_scalar_prefetch=1,
        grid=(NG,),
        in_specs=[
            pl.BlockSpec((SM, NUM_IN // 2), lambda b, s: (b, 0)),
            pl.BlockSpec((E, NUM_IN, TOTAL_OUT), lambda b, s: (0, 0, 0)),
            pl.BlockSpec((E, TOTAL_OUT), lambda b, s: (0, 0)),
        ],
        out_specs=[
            pl.BlockSpec((SM, HALF_OUT), lambda b, s: (b, 0)),
            pl.BlockSpec((SM, HALF_OUT), lambda b, s: (b, 0)),
        ],
    )
    return pl.pallas_call(
        _gemm_body,
        grid_spec=grid_spec,
        out_shape=[jax.ShapeDtypeStruct((CAP, HALF_OUT), jnp.float32),
                   jax.ShapeDtypeStruct((CAP, HALF_OUT), jnp.float32)],
    )(bexp, xs, expert_W_bf16, expert_b)


def _gath_body(pos_hbm, y1_hbm, y2_hbm, o1_hbm, o2_hbm,
               pv_v, y1_v, y2_v, sem, wsem):
    wid = _wid()
    pltpu.sync_copy(pos_hbm.at[pl.ds(wid * RPT, RPT)], pv_v)
    g1 = [pltpu.async_copy(y1_hbm.at[pv_v.at[j]],
                           y1_v.at[pl.ds(j * 128, 128)], sem)
          for j in range(RPT)]
    g2 = [pltpu.async_copy(y2_hbm.at[pv_v.at[j]],
                           y2_v.at[pl.ds(j * 128, 128)], sem)
          for j in range(RPT)]
    for c in g1:
        c.wait()
    w1 = pltpu.async_copy(y1_v, o1_hbm.at[pl.ds(wid * TPT, TPT)], wsem)
    for c in g2:
        c.wait()
    w2 = pltpu.async_copy(y2_v, o2_hbm.at[pl.ds(wid * TPT, TPT)], wsem)
    w1.wait()
    w2.wait()


def _gather_y(pos, y1, y2):
    f = functools.partial(
        pl.kernel,
        out_type=[jax.ShapeDtypeStruct((N_TOK, HALF_OUT), jnp.float32),
                  jax.ShapeDtypeStruct((N_TOK, HALF_OUT), jnp.float32)],
        mesh=_mesh(),
        scratch_types=[
            pltpu.VMEM((RPT, 128), jnp.int32),
            pltpu.VMEM((TPT, HALF_OUT), jnp.float32),
            pltpu.VMEM((TPT, HALF_OUT), jnp.float32),
            pltpu.SemaphoreType.DMA,
            pltpu.SemaphoreType.DMA,
        ],
        compiler_params=_SC_PARAMS,
    )(_gath_body)
    return f(pos, y1, y2)


def kernel(x, type_embeddings, atom_types, gate_W, gate_b, expert_W, expert_b):
    atom_types = atom_types.astype(jnp.int32)
    em = _expert_map(type_embeddings, gate_W, gate_b)
    xbf = x.astype(jnp.bfloat16)
    xi = lax.bitcast_convert_type(
        jnp.stack([xbf[:, :NUM_IN // 2], xbf[:, NUM_IN // 2:]], axis=-1),
        jnp.int32)
    pos, bexp, xs = _sort_scatter(em, atom_types, xi)
    y1, y2 = _gemm(bexp, xs, expert_W.astype(jnp.bfloat16), expert_b)
    o1, o2 = _gather_y(pos, y1, y2)
    return (o1, o2)


# speedup vs baseline: 1.6101x; 1.0531x over previous
import functools

import jax
import jax.numpy as jnp
from jax import lax
from jax.experimental import pallas as pl
from jax.experimental.pallas import tpu as pltpu
from jax.experimental.pallas import tpu_sc as plsc

N_TOK = 8192
NUM_IN = 256
TOTAL_OUT = 256
HALF_OUT = 128
E = 16
NTYPES = 128
TEBD = 64

TM = 256
CAP = 12288
NB = CAP // TM
TM_SHIFT = TM.bit_length() - 1

NC, NS = 2, 16
NW = NC * NS
TPT = N_TOK // NW
RPT = TPT // 128
NV = N_TOK // 16


def _wid():
    return lax.axis_index("s") * NC + lax.axis_index("c")


def _mesh():
    return plsc.VectorSubcoreMesh(core_axis_name="c", subcore_axis_name="s")


_SC_PARAMS = pltpu.CompilerParams(needs_layout_passes=False)


def _emap_body(te_ref, gw_ref, gb_ref, em_ref):
    logits = jnp.dot(te_ref[...], gw_ref[...],
                     preferred_element_type=jnp.float32) + gb_ref[...]
    em_ref[...] = jnp.argmax(logits, axis=1).astype(jnp.int32)[None, :]


def _expert_map(type_embeddings, gate_W, gate_b):
    return pl.pallas_call(
        _emap_body,
        out_shape=jax.ShapeDtypeStruct((1, NTYPES), jnp.int32),
    )(type_embeddings, gate_W, gate_b.reshape(1, E))


def _sort_body(em_hbm, at_hbm, x_hbm, pos_hbm, bexp_hbm, xs_hbm,
               em_v, at_v, ev_v, hist_v, ctr_v, pos2_v, bexp_v, xv_v,
               sem, xsem):
    wid = _wid()
    lane = lax.broadcasted_iota(jnp.int32, (E,), 0)
    ones = jnp.ones((E,), jnp.int32)

    xload = pltpu.async_copy(x_hbm.at[pl.ds(wid * TPT, TPT)], xv_v, xsem)
    pltpu.sync_copy(em_hbm.at[0], em_v)
    pltpu.sync_copy(at_hbm, at_v)

    for r in range(8):
        hist_v[pl.ds(r * 16, 16)] = jnp.zeros((E,), jnp.int32)

    def _acc8(i, carry):
        for r in range(8):
            tv = at_v[pl.ds((i * 8 + r) * 16, 16)]
            evv = plsc.load_gather(em_v, [tv])
            plsc.addupdate_scatter(hist_v, [r * 16 + evv], ones)
        return carry

    def _hsum():
        h = jnp.zeros((E,), jnp.int32)
        for r in range(8):
            h = h + hist_v[pl.ds(r * 16, 16)]
        return h

    lax.fori_loop(0, wid * (TPT // 128), _acc8, 0)
    bef = _hsum()
    base = wid * (TPT // 16)

    def _acc_own(i, carry):
        tv = at_v[pl.ds((base + i) * 16, 16)]
        evv = plsc.load_gather(em_v, [tv])
        ev_v[pl.ds(i * 16, 16)] = evv
        plsc.addupdate_scatter(hist_v, [(i & 7) * 16 + evv], ones)
        return carry
    lax.fori_loop(0, TPT // 16, _acc_own, 0)
    lax.fori_loop((base + TPT // 16) // 8, NV // 8, _acc8, 0)
    g = _hsum()

    padded = ((g + (TM - 1)) >> TM_SHIFT) << TM_SHIFT
    offs = plsc.cumsum(padded) - padded
    start = offs + bef

    @pl.when(wid == 0)
    def _():
        for k in range(NB // 16):
            bv = (lane + 16 * k) * TM
            be = jnp.zeros((E,), jnp.int32)
            for e in range(E):
                be = jnp.where((bv >= offs[e]) & (bv < offs[e] + padded[e]),
                               e, be)
            bexp_v[pl.ds(16 * k, 16)] = be
        pltpu.sync_copy(bexp_v, bexp_hbm)

    ctr_v[...] = start

    def _rank(i, carry):
        evv = ev_v[pl.ds(i * 16, 16)]
        base_ = plsc.load_gather(ctr_v, [evv])
        rk, _ = plsc.scan_count(evv)
        posv = jnp.minimum(base_ + rk - 1, CAP - 1)
        li = i * 16 + lane
        plsc.store_scatter(pos2_v, [li >> 7, li & 127], posv)
        plsc.addupdate_scatter(ctr_v, [evv], ones)
        return carry
    lax.fori_loop(0, TPT // 16, _rank, 0)

    pltpu.sync_copy(pos2_v, pos_hbm.at[pl.ds(wid * RPT, RPT)])
    xload.wait()
    cps = [pltpu.async_copy(xv_v.at[pl.ds(j * 128, 128)],
                            xs_hbm.at[pos2_v.at[j]], sem)
           for j in range(RPT)]
    for c in cps:
        c.wait()


def _sort_scatter(em, atom_types, x):
    f = functools.partial(
        pl.kernel,
        out_type=[jax.ShapeDtypeStruct((N_TOK // 128, 128), jnp.int32),
                  jax.ShapeDtypeStruct((NB,), jnp.int32),
                  jax.ShapeDtypeStruct((CAP, NUM_IN), jnp.float32)],
        mesh=_mesh(),
        scratch_types=[
            pltpu.VMEM((NTYPES,), jnp.int32),
            pltpu.VMEM((N_TOK,), jnp.int32),
            pltpu.VMEM((TPT,), jnp.int32),
            pltpu.VMEM((8 * E,), jnp.int32),
            pltpu.VMEM((E,), jnp.int32),
            pltpu.VMEM((RPT, 128), jnp.int32),
            pltpu.VMEM((NB,), jnp.int32),
            pltpu.VMEM((TPT, NUM_IN), jnp.float32),
            pltpu.SemaphoreType.DMA,
            pltpu.SemaphoreType.DMA,
        ],
        compiler_params=_SC_PARAMS,
    )(_sort_body)
    return f(em, atom_types, x)


BPS = 4
SM = TM * BPS
NG = CAP // SM


def _gemm_body(bexp_ref, xs_ref, w_ref, b_ref, y1_ref, y2_ref):
    gidx = pl.program_id(0)
    for k in range(BPS):
        e = bexp_ref[gidx * BPS + k]
        xb = xs_ref[pl.ds(k * TM, TM), :].astype(jnp.bfloat16)
        w = w_ref[pl.ds(e, 1), :, :][0]
        y = jnp.tanh(
            jnp.dot(xb, w, preferred_element_type=jnp.float32)
            + b_ref[pl.ds(e, 1), :])
        y1_ref[pl.ds(k * TM, TM), :] = y[:, :HALF_OUT]
        y2_ref[pl.ds(k * TM, TM), :] = y[:, HALF_OUT:]


def _gemm(bexp, xs, expert_W_bf16, expert_b):
    grid_spec = pltpu.PrefetchScalarGridSpec(
        num_scalar_prefetch=1,
        grid=(NG,),
        in_specs=[
            pl.BlockSpec((SM, NUM_IN), lambda b, s: (b, 0)),
            pl.BlockSpec((E, NUM_IN, TOTAL_OUT), lambda b, s: (0, 0, 0)),
            pl.BlockSpec((E, TOTAL_OUT), lambda b, s: (0, 0)),
        ],
        out_specs=[
            pl.BlockSpec((SM, HALF_OUT), lambda b, s: (b, 0)),
            pl.BlockSpec((SM, HALF_OUT), lambda b, s: (b, 0)),
        ],
    )
    return pl.pallas_call(
        _gemm_body,
        grid_spec=grid_spec,
        out_shape=[jax.ShapeDtypeStruct((CAP, HALF_OUT), jnp.float32),
                   jax.ShapeDtypeStruct((CAP, HALF_OUT), jnp.float32)],
    )(bexp, xs, expert_W_bf16, expert_b)


def _gath_body(pos_hbm, y1_hbm, y2_hbm, o1_hbm, o2_hbm,
               pv_v, y1_v, y2_v, sem, wsem):
    wid = _wid()
    pltpu.sync_copy(pos_hbm.at[pl.ds(wid * RPT, RPT)], pv_v)
    g1 = [pltpu.async_copy(y1_hbm.at[pv_v.at[j]],
                           y1_v.at[pl.ds(j * 128, 128)], sem)
          for j in range(RPT)]
    g2 = [pltpu.async_copy(y2_hbm.at[pv_v.at[j]],
                           y2_v.at[pl.ds(j * 128, 128)], sem)
          for j in range(RPT)]
    for c in g1:
        c.wait()
    w1 = pltpu.async_copy(y1_v, o1_hbm.at[pl.ds(wid * TPT, TPT)], wsem)
    for c in g2:
        c.wait()
    w2 = pltpu.async_copy(y2_v, o2_hbm.at[pl.ds(wid * TPT, TPT)], wsem)
    w1.wait()
    w2.wait()


def _gather_y(pos, y1, y2):
    f = functools.partial(
        pl.kernel,
        out_type=[jax.ShapeDtypeStruct((N_TOK, HALF_OUT), jnp.float32),
                  jax.ShapeDtypeStruct((N_TOK, HALF_OUT), jnp.float32)],
        mesh=_mesh(),
        scratch_types=[
            pltpu.VMEM((RPT, 128), jnp.int32),
            pltpu.VMEM((TPT, HALF_OUT), jnp.float32),
            pltpu.VMEM((TPT, HALF_OUT), jnp.float32),
            pltpu.SemaphoreType.DMA,
            pltpu.SemaphoreType.DMA,
        ],
        compiler_params=_SC_PARAMS,
    )(_gath_body)
    return f(pos, y1, y2)


def kernel(x, type_embeddings, atom_types, gate_W, gate_b, expert_W, expert_b):
    atom_types = atom_types.astype(jnp.int32)
    em = _expert_map(type_embeddings, gate_W, gate_b)
    pos, bexp, xs = _sort_scatter(em, atom_types, x)
    y1, y2 = _gemm(bexp, xs, expert_W.astype(jnp.bfloat16), expert_b)
    o1, o2 = _gather_y(pos, y1, y2)
    return (o1, o2)


# speedup vs baseline: 1.6684x; 1.0362x over previous
import functools

import jax
import jax.numpy as jnp
from jax import lax
from jax.experimental import pallas as pl
from jax.experimental.pallas import tpu as pltpu
from jax.experimental.pallas import tpu_sc as plsc

N_TOK = 8192
NUM_IN = 256
TOTAL_OUT = 256
HALF_OUT = 128
E = 16
NTYPES = 128
TEBD = 64

TM = 128
CAP = 10240
NB = CAP // TM
TM_SHIFT = TM.bit_length() - 1

NC, NS = 2, 16
NW = NC * NS
TPT = N_TOK // NW
RPT = TPT // 128
NV = N_TOK // 16


def _wid():
    return lax.axis_index("s") * NC + lax.axis_index("c")


def _mesh():
    return plsc.VectorSubcoreMesh(core_axis_name="c", subcore_axis_name="s")


_SC_PARAMS = pltpu.CompilerParams(needs_layout_passes=False)


def _emap_body(te_ref, gw_ref, gb_ref, em_ref):
    logits = jnp.dot(te_ref[...], gw_ref[...],
                     preferred_element_type=jnp.float32) + gb_ref[...]
    em_ref[...] = jnp.argmax(logits, axis=1).astype(jnp.int32)[None, :]


def _expert_map(type_embeddings, gate_W, gate_b):
    return pl.pallas_call(
        _emap_body,
        out_shape=jax.ShapeDtypeStruct((1, NTYPES), jnp.int32),
    )(type_embeddings, gate_W, gate_b.reshape(1, E))


def _sort_body(em_hbm, at_hbm, x_hbm, pos_hbm, bexp_hbm, xs_hbm,
               em_v, at_v, ev_v, hist_v, ctr_v, pos2_v, bexp_v, xv_v,
               sem, xsem):
    wid = _wid()
    lane = lax.broadcasted_iota(jnp.int32, (E,), 0)
    ones = jnp.ones((E,), jnp.int32)

    xload = pltpu.async_copy(x_hbm.at[pl.ds(wid * TPT, TPT)], xv_v, xsem)
    pltpu.sync_copy(em_hbm.at[0], em_v)
    pltpu.sync_copy(at_hbm, at_v)

    for r in range(8):
        hist_v[pl.ds(r * 16, 16)] = jnp.zeros((E,), jnp.int32)

    def _acc8(i, carry):
        for r in range(8):
            tv = at_v[pl.ds((i * 8 + r) * 16, 16)]
            evv = plsc.load_gather(em_v, [tv])
            plsc.addupdate_scatter(hist_v, [r * 16 + evv], ones)
        return carry

    def _hsum():
        h = jnp.zeros((E,), jnp.int32)
        for r in range(8):
            h = h + hist_v[pl.ds(r * 16, 16)]
        return h

    lax.fori_loop(0, wid * (TPT // 128), _acc8, 0)
    bef = _hsum()
    base = wid * (TPT // 16)

    def _acc_own(i, carry):
        tv = at_v[pl.ds((base + i) * 16, 16)]
        evv = plsc.load_gather(em_v, [tv])
        ev_v[pl.ds(i * 16, 16)] = evv
        plsc.addupdate_scatter(hist_v, [(i & 7) * 16 + evv], ones)
        return carry
    lax.fori_loop(0, TPT // 16, _acc_own, 0)
    lax.fori_loop((base + TPT // 16) // 8, NV // 8, _acc8, 0)
    g = _hsum()

    padded = ((g + (TM - 1)) >> TM_SHIFT) << TM_SHIFT
    offs = plsc.cumsum(padded) - padded
    start = offs + bef

    @pl.when(wid == 0)
    def _():
        for k in range(NB // 16):
            bv = (lane + 16 * k) * TM
            be = jnp.zeros((E,), jnp.int32)
            for e in range(E):
                be = jnp.where((bv >= offs[e]) & (bv < offs[e] + padded[e]),
                               e, be)
            bexp_v[pl.ds(16 * k, 16)] = be
        pltpu.sync_copy(bexp_v, bexp_hbm)

    ctr_v[...] = start

    def _rank(i, carry):
        evv = ev_v[pl.ds(i * 16, 16)]
        base_ = plsc.load_gather(ctr_v, [evv])
        rk, _ = plsc.scan_count(evv)
        posv = jnp.minimum(base_ + rk - 1, CAP - 1)
        li = i * 16 + lane
        plsc.store_scatter(pos2_v, [li >> 7, li & 127], posv)
        plsc.addupdate_scatter(ctr_v, [evv], ones)
        return carry
    lax.fori_loop(0, TPT // 16, _rank, 0)

    pltpu.sync_copy(pos2_v, pos_hbm.at[pl.ds(wid * RPT, RPT)])
    xload.wait()
    cps = [pltpu.async_copy(xv_v.at[pl.ds(j * 128, 128)],
                            xs_hbm.at[pos2_v.at[j]], sem)
           for j in range(RPT)]
    for c in cps:
        c.wait()


def _sort_scatter(em, atom_types, x):
    f = functools.partial(
        pl.kernel,
        out_type=[jax.ShapeDtypeStruct((N_TOK // 128, 128), jnp.int32),
                  jax.ShapeDtypeStruct((NB,), jnp.int32),
                  jax.ShapeDtypeStruct((CAP, NUM_IN), jnp.float32)],
        mesh=_mesh(),
        scratch_types=[
            pltpu.VMEM((NTYPES,), jnp.int32),
            pltpu.VMEM((N_TOK,), jnp.int32),
            pltpu.VMEM((TPT,), jnp.int32),
            pltpu.VMEM((8 * E,), jnp.int32),
            pltpu.VMEM((E,), jnp.int32),
            pltpu.VMEM((RPT, 128), jnp.int32),
            pltpu.VMEM((NB,), jnp.int32),
            pltpu.VMEM((TPT, NUM_IN), jnp.float32),
            pltpu.SemaphoreType.DMA,
            pltpu.SemaphoreType.DMA,
        ],
        compiler_params=_SC_PARAMS,
    )(_sort_body)
    return f(em, atom_types, x)


BPS = 8
SM = TM * BPS
NG = CAP // SM


def _gemm_body(bexp_ref, xs_ref, w_ref, b_ref, y1_ref, y2_ref):
    gidx = pl.program_id(0)
    for k in range(BPS):
        e = bexp_ref[gidx * BPS + k]
        xb = xs_ref[pl.ds(k * TM, TM), :].astype(jnp.bfloat16)
        w = w_ref[pl.ds(e, 1), :, :][0]
        y = jnp.tanh(
            jnp.dot(xb, w, preferred_element_type=jnp.float32)
            + b_ref[pl.ds(e, 1), :])
        y1_ref[pl.ds(k * TM, TM), :] = y[:, :HALF_OUT]
        y2_ref[pl.ds(k * TM, TM), :] = y[:, HALF_OUT:]


def _gemm(bexp, xs, expert_W_bf16, expert_b):
    grid_spec = pltpu.PrefetchScalarGridSpec(
        num_scalar_prefetch=1,
        grid=(NG,),
        in_specs=[
            pl.BlockSpec((SM, NUM_IN), lambda b, s: (b, 0)),
            pl.BlockSpec((E, NUM_IN, TOTAL_OUT), lambda b, s: (0, 0, 0)),
            pl.BlockSpec((E, TOTAL_OUT), lambda b, s: (0, 0)),
        ],
        out_specs=[
            pl.BlockSpec((SM, HALF_OUT), lambda b, s: (b, 0)),
            pl.BlockSpec((SM, HALF_OUT), lambda b, s: (b, 0)),
        ],
    )
    return pl.pallas_call(
        _gemm_body,
        grid_spec=grid_spec,
        out_shape=[jax.ShapeDtypeStruct((CAP, HALF_OUT), jnp.float32),
                   jax.ShapeDtypeStruct((CAP, HALF_OUT), jnp.float32)],
    )(bexp, xs, expert_W_bf16, expert_b)


def _gath_body(pos_hbm, y1_hbm, y2_hbm, o1_hbm, o2_hbm,
               pv_v, y1_v, y2_v, sem, wsem):
    wid = _wid()
    pltpu.sync_copy(pos_hbm.at[pl.ds(wid * RPT, RPT)], pv_v)
    g1 = [pltpu.async_copy(y1_hbm.at[pv_v.at[j]],
                           y1_v.at[pl.ds(j * 128, 128)], sem)
          for j in range(RPT)]
    g2 = [pltpu.async_copy(y2_hbm.at[pv_v.at[j]],
                           y2_v.at[pl.ds(j * 128, 128)], sem)
          for j in range(RPT)]
    for c in g1:
        c.wait()
    w1 = pltpu.async_copy(y1_v, o1_hbm.at[pl.ds(wid * TPT, TPT)], wsem)
    for c in g2:
        c.wait()
    w2 = pltpu.async_copy(y2_v, o2_hbm.at[pl.ds(wid * TPT, TPT)], wsem)
    w1.wait()
    w2.wait()


def _gather_y(pos, y1, y2):
    f = functools.partial(
        pl.kernel,
        out_type=[jax.ShapeDtypeStruct((N_TOK, HALF_OUT), jnp.float32),
                  jax.ShapeDtypeStruct((N_TOK, HALF_OUT), jnp.float32)],
        mesh=_mesh(),
        scratch_types=[
            pltpu.VMEM((RPT, 128), jnp.int32),
            pltpu.VMEM((TPT, HALF_OUT), jnp.float32),
            pltpu.VMEM((TPT, HALF_OUT), jnp.float32),
            pltpu.SemaphoreType.DMA,
            pltpu.SemaphoreType.DMA,
        ],
        compiler_params=_SC_PARAMS,
    )(_gath_body)
    return f(pos, y1, y2)


def kernel(x, type_embeddings, atom_types, gate_W, gate_b, expert_W, expert_b):
    atom_types = atom_types.astype(jnp.int32)
    em = _expert_map(type_embeddings, gate_W, gate_b)
    pos, bexp, xs = _sort_scatter(em, atom_types, x)
    y1, y2 = _gemm(bexp, xs, expert_W.astype(jnp.bfloat16), expert_b)
    o1, o2 = _gather_y(pos, y1, y2)
    return (o1, o2)


# speedup vs baseline: 1.6779x; 1.0057x over previous
import functools

import jax
import jax.numpy as jnp
from jax import lax
from jax.experimental import pallas as pl
from jax.experimental.pallas import tpu as pltpu
from jax.experimental.pallas import tpu_sc as plsc

N_TOK = 8192
NUM_IN = 256
TOTAL_OUT = 256
HALF_OUT = 128
E = 16
NTYPES = 128
TEBD = 64

TM = 128
CAP = 10240
NB = CAP // TM
TM_SHIFT = TM.bit_length() - 1

NC, NS = 2, 16
NW = NC * NS
TPT = N_TOK // NW
RPT = TPT // 128
NV = N_TOK // 16


def _wid():
    return lax.axis_index("s") * NC + lax.axis_index("c")


def _mesh():
    return plsc.VectorSubcoreMesh(core_axis_name="c", subcore_axis_name="s")


_SC_PARAMS = pltpu.CompilerParams(needs_layout_passes=False)


def _emap_body(te_ref, gw_ref, gb_ref, em_ref):
    logits = jnp.dot(te_ref[...], gw_ref[...],
                     preferred_element_type=jnp.float32) + gb_ref[...]
    em_ref[...] = jnp.argmax(logits, axis=1).astype(jnp.int32)[None, :]


def _expert_map(type_embeddings, gate_W, gate_b):
    return pl.pallas_call(
        _emap_body,
        out_shape=jax.ShapeDtypeStruct((1, NTYPES), jnp.int32),
    )(type_embeddings, gate_W, gate_b.reshape(1, E))


def _sort_body(em_hbm, at_hbm, x_hbm, pos_hbm, bexp_hbm, xs_hbm,
               em_v, at_v, ev_v, hist_v, ctr_v, pos2_v, bexp_v, xv_v,
               sem, xsem):
    wid = _wid()
    lane = lax.broadcasted_iota(jnp.int32, (E,), 0)
    ones = jnp.ones((E,), jnp.int32)

    xload = pltpu.async_copy(x_hbm.at[pl.ds(wid * TPT, TPT)], xv_v, xsem)
    pltpu.sync_copy(em_hbm.at[0], em_v)
    pltpu.sync_copy(at_hbm, at_v)

    for r in range(8):
        hist_v[pl.ds(r * 16, 16)] = jnp.zeros((E,), jnp.int32)

    def _acc8(i, carry):
        for r in range(8):
            tv = at_v[pl.ds((i * 8 + r) * 16, 16)]
            evv = plsc.load_gather(em_v, [tv])
            plsc.addupdate_scatter(hist_v, [r * 16 + evv], ones)
        return carry

    def _hsum():
        h = jnp.zeros((E,), jnp.int32)
        for r in range(8):
            h = h + hist_v[pl.ds(r * 16, 16)]
        return h

    lax.fori_loop(0, wid * (TPT // 128), _acc8, 0)
    bef = _hsum()
    base = wid * (TPT // 16)

    def _acc_own(i, carry):
        tv = at_v[pl.ds((base + i) * 16, 16)]
        evv = plsc.load_gather(em_v, [tv])
        ev_v[pl.ds(i * 16, 16)] = evv
        plsc.addupdate_scatter(hist_v, [(i & 7) * 16 + evv], ones)
        return carry
    lax.fori_loop(0, TPT // 16, _acc_own, 0)
    lax.fori_loop((base + TPT // 16) // 8, NV // 8, _acc8, 0)
    g = _hsum()

    padded = ((g + (TM - 1)) >> TM_SHIFT) << TM_SHIFT
    offs = plsc.cumsum(padded) - padded
    start = offs + bef

    @pl.when(wid == 0)
    def _():
        for k in range(NB // 16):
            bv = (lane + 16 * k) * TM
            be = jnp.zeros((E,), jnp.int32)
            for e in range(E):
                be = jnp.where((bv >= offs[e]) & (bv < offs[e] + padded[e]),
                               e, be)
            bexp_v[pl.ds(16 * k, 16)] = be
        pltpu.sync_copy(bexp_v, bexp_hbm)

    ctr_v[...] = start

    def _rank(i, carry):
        evv = ev_v[pl.ds(i * 16, 16)]
        base_ = plsc.load_gather(ctr_v, [evv])
        rk, _ = plsc.scan_count(evv)
        posv = jnp.minimum(base_ + rk - 1, CAP - 1)
        li = i * 16 + lane
        plsc.store_scatter(pos2_v, [li >> 7, li & 127], posv)
        plsc.addupdate_scatter(ctr_v, [evv], ones)
        return carry

    cps = []
    for j in range(RPT):
        lax.fori_loop(j * 8, (j + 1) * 8, _rank, 0)
        if j == 0:
            xload.wait()
        cps.append(pltpu.async_copy(xv_v.at[pl.ds(j * 128, 128)],
                                    xs_hbm.at[pos2_v.at[j]], sem))
    cps.append(pltpu.async_copy(pos2_v, pos_hbm.at[pl.ds(wid * RPT, RPT)],
                                xsem))
    for c in cps:
        c.wait()


def _sort_scatter(em, atom_types, x):
    f = functools.partial(
        pl.kernel,
        out_type=[jax.ShapeDtypeStruct((N_TOK // 128, 128), jnp.int32),
                  jax.ShapeDtypeStruct((NB,), jnp.int32),
                  jax.ShapeDtypeStruct((CAP, NUM_IN), jnp.float32)],
        mesh=_mesh(),
        scratch_types=[
            pltpu.VMEM((NTYPES,), jnp.int32),
            pltpu.VMEM((N_TOK,), jnp.int32),
            pltpu.VMEM((TPT,), jnp.int32),
            pltpu.VMEM((8 * E,), jnp.int32),
            pltpu.VMEM((E,), jnp.int32),
            pltpu.VMEM((RPT, 128), jnp.int32),
            pltpu.VMEM((NB,), jnp.int32),
            pltpu.VMEM((TPT, NUM_IN), jnp.float32),
            pltpu.SemaphoreType.DMA,
            pltpu.SemaphoreType.DMA,
        ],
        compiler_params=_SC_PARAMS,
    )(_sort_body)
    return f(em, atom_types, x)


BPS = 8
SM = TM * BPS
NG = CAP // SM


def _gemm_body(bexp_ref, xs_ref, w_ref, b_ref, y1_ref, y2_ref):
    gidx = pl.program_id(0)
    for k in range(BPS):
        e = bexp_ref[gidx * BPS + k]
        xb = xs_ref[pl.ds(k * TM, TM), :].astype(jnp.bfloat16)
        w = w_ref[pl.ds(e, 1), :, :][0]
        y = jnp.tanh(
            jnp.dot(xb, w, preferred_element_type=jnp.float32)
            + b_ref[pl.ds(e, 1), :])
        y1_ref[pl.ds(k * TM, TM), :] = y[:, :HALF_OUT]
        y2_ref[pl.ds(k * TM, TM), :] = y[:, HALF_OUT:]


def _gemm(bexp, xs, expert_W_bf16, expert_b):
    grid_spec = pltpu.PrefetchScalarGridSpec(
        num_scalar_prefetch=1,
        grid=(NG,),
        in_specs=[
            pl.BlockSpec((SM, NUM_IN), lambda b, s: (b, 0)),
            pl.BlockSpec((E, NUM_IN, TOTAL_OUT), lambda b, s: (0, 0, 0)),
            pl.BlockSpec((E, TOTAL_OUT), lambda b, s: (0, 0)),
        ],
        out_specs=[
            pl.BlockSpec((SM, HALF_OUT), lambda b, s: (b, 0)),
            pl.BlockSpec((SM, HALF_OUT), lambda b, s: (b, 0)),
        ],
    )
    return pl.pallas_call(
        _gemm_body,
        grid_spec=grid_spec,
        out_shape=[jax.ShapeDtypeStruct((CAP, HALF_OUT), jnp.float32),
                   jax.ShapeDtypeStruct((CAP, HALF_OUT), jnp.float32)],
    )(bexp, xs, expert_W_bf16, expert_b)


def _gath_body(pos_hbm, y1_hbm, y2_hbm, o1_hbm, o2_hbm,
               pv_v, y1_v, y2_v, sem, wsem):
    wid = _wid()
    pltpu.sync_copy(pos_hbm.at[pl.ds(wid * RPT, RPT)], pv_v)
    g1 = [pltpu.async_copy(y1_hbm.at[pv_v.at[j]],
                           y1_v.at[pl.ds(j * 128, 128)], sem)
          for j in range(RPT)]
    g2 = [pltpu.async_copy(y2_hbm.at[pv_v.at[j]],
                           y2_v.at[pl.ds(j * 128, 128)], sem)
          for j in range(RPT)]
    ws = []
    for j in range(RPT):
        g1[j].wait()
        ws.append(pltpu.async_copy(
            y1_v.at[pl.ds(j * 128, 128)],
            o1_hbm.at[pl.ds(wid * TPT + j * 128, 128)], wsem))
        g2[j].wait()
        ws.append(pltpu.async_copy(
            y2_v.at[pl.ds(j * 128, 128)],
            o2_hbm.at[pl.ds(wid * TPT + j * 128, 128)], wsem))
    for w in ws:
        w.wait()


def _gather_y(pos, y1, y2):
    f = functools.partial(
        pl.kernel,
        out_type=[jax.ShapeDtypeStruct((N_TOK, HALF_OUT), jnp.float32),
                  jax.ShapeDtypeStruct((N_TOK, HALF_OUT), jnp.float32)],
        mesh=_mesh(),
        scratch_types=[
            pltpu.VMEM((RPT, 128), jnp.int32),
            pltpu.VMEM((TPT, HALF_OUT), jnp.float32),
            pltpu.VMEM((TPT, HALF_OUT), jnp.float32),
            pltpu.SemaphoreType.DMA,
            pltpu.SemaphoreType.DMA,
        ],
        compiler_params=_SC_PARAMS,
    )(_gath_body)
    return f(pos, y1, y2)


def kernel(x, type_embeddings, atom_types, gate_W, gate_b, expert_W, expert_b):
    atom_types = atom_types.astype(jnp.int32)
    em = _expert_map(type_embeddings, gate_W, gate_b)
    pos, bexp, xs = _sort_scatter(em, atom_types, x)
    y1, y2 = _gemm(bexp, xs, expert_W.astype(jnp.bfloat16), expert_b)
    o1, o2 = _gather_y(pos, y1, y2)
    return (o1, o2)


# speedup vs baseline: 1.6793x; 1.0008x over previous
import functools

import jax
import jax.numpy as jnp
from jax import lax
from jax.experimental import pallas as pl
from jax.experimental.pallas import tpu as pltpu
from jax.experimental.pallas import tpu_sc as plsc

N_TOK = 8192
NUM_IN = 256
TOTAL_OUT = 256
HALF_OUT = 128
E = 16
NTYPES = 128
TEBD = 64

TM = 128
CAP = 10240
NB = CAP // TM
TM_SHIFT = TM.bit_length() - 1

NC, NS = 2, 16
NW = NC * NS
TPT = N_TOK // NW
RPT = TPT // 128
NV = N_TOK // 16


def _wid():
    return lax.axis_index("s") * NC + lax.axis_index("c")


def _mesh():
    return plsc.VectorSubcoreMesh(core_axis_name="c", subcore_axis_name="s")


_SC_PARAMS = pltpu.CompilerParams(needs_layout_passes=False)


def _emap_body(te_ref, gw_ref, gb_ref, em_ref):
    logits = jnp.dot(te_ref[...], gw_ref[...],
                     preferred_element_type=jnp.float32) + gb_ref[...]
    em_ref[...] = jnp.argmax(logits, axis=1).astype(jnp.int32)[None, :]


def _expert_map(type_embeddings, gate_W, gate_b):
    return pl.pallas_call(
        _emap_body,
        out_shape=jax.ShapeDtypeStruct((1, NTYPES), jnp.int32),
    )(type_embeddings, gate_W, gate_b.reshape(1, E))


def _sort_body(em_hbm, at_hbm, x_hbm, pos_hbm, bexp_hbm, xs_hbm,
               em_v, at_v, ev_v, hist_v, ctr_v, pos2_v, bexp_v, xv_v,
               sem, xsem):
    wid = _wid()
    lane = lax.broadcasted_iota(jnp.int32, (E,), 0)
    ones = jnp.ones((E,), jnp.int32)

    xload = pltpu.async_copy(x_hbm.at[pl.ds(wid * TPT, TPT)], xv_v, xsem)
    pltpu.sync_copy(em_hbm.at[0], em_v)
    pltpu.sync_copy(at_hbm, at_v)

    for r in range(8):
        hist_v[pl.ds(r * 16, 16)] = jnp.zeros((E,), jnp.int32)

    def _acc8(i, carry):
        for r in range(8):
            tv = at_v[pl.ds((i * 8 + r) * 16, 16)]
            evv = plsc.load_gather(em_v, [tv])
            plsc.addupdate_scatter(hist_v, [r * 16 + evv], ones)
        return carry

    def _hsum():
        h = jnp.zeros((E,), jnp.int32)
        for r in range(8):
            h = h + hist_v[pl.ds(r * 16, 16)]
        return h

    lax.fori_loop(0, wid * (TPT // 128), _acc8, 0)
    bef = _hsum()
    base = wid * (TPT // 16)

    def _acc_own(i, carry):
        tv = at_v[pl.ds((base + i) * 16, 16)]
        evv = plsc.load_gather(em_v, [tv])
        ev_v[pl.ds(i * 16, 16)] = evv
        plsc.addupdate_scatter(hist_v, [(i & 7) * 16 + evv], ones)
        return carry
    lax.fori_loop(0, TPT // 16, _acc_own, 0)
    lax.fori_loop((base + TPT // 16) // 8, NV // 8, _acc8, 0)
    g = _hsum()

    padded = ((g + (TM - 1)) >> TM_SHIFT) << TM_SHIFT
    offs = plsc.cumsum(padded) - padded
    start = offs + bef

    @pl.when(wid == 0)
    def _():
        for k in range(NB // 16):
            bv = (lane + 16 * k) * TM
            be = jnp.zeros((E,), jnp.int32)
            for e in range(E):
                be = jnp.where((bv >= offs[e]) & (bv < offs[e] + padded[e]),
                               e, be)
            bexp_v[pl.ds(16 * k, 16)] = be
        pltpu.sync_copy(bexp_v, bexp_hbm)

    ctr_v[...] = start

    def _rank(i, carry):
        evv = ev_v[pl.ds(i * 16, 16)]
        base_ = plsc.load_gather(ctr_v, [evv])
        rk, _ = plsc.scan_count(evv)
        posv = jnp.minimum(base_ + rk - 1, CAP - 1)
        li = i * 16 + lane
        plsc.store_scatter(pos2_v, [li >> 7, li & 127], posv)
        plsc.addupdate_scatter(ctr_v, [evv], ones)
        return carry

    cps = []
    for j in range(RPT):
        lax.fori_loop(j * 8, (j + 1) * 8, _rank, 0)
        if j == 0:
            xload.wait()
        cps.append(pltpu.async_copy(xv_v.at[pl.ds(j * 128, 128)],
                                    xs_hbm.at[pos2_v.at[j]], sem))
    cps.append(pltpu.async_copy(pos2_v, pos_hbm.at[pl.ds(wid * RPT, RPT)],
                                xsem))
    for c in cps:
        c.wait()


def _sort_scatter(em, atom_types, x):
    f = functools.partial(
        pl.kernel,
        out_type=[jax.ShapeDtypeStruct((N_TOK // 128, 128), jnp.int32),
                  jax.ShapeDtypeStruct((NB,), jnp.int32),
                  jax.ShapeDtypeStruct((CAP, NUM_IN), jnp.float32)],
        mesh=_mesh(),
        scratch_types=[
            pltpu.VMEM((NTYPES,), jnp.int32),
            pltpu.VMEM((N_TOK,), jnp.int32),
            pltpu.VMEM((TPT,), jnp.int32),
            pltpu.VMEM((8 * E,), jnp.int32),
            pltpu.VMEM((E,), jnp.int32),
            pltpu.VMEM((RPT, 128), jnp.int32),
            pltpu.VMEM((NB,), jnp.int32),
            pltpu.VMEM((TPT, NUM_IN), jnp.float32),
            pltpu.SemaphoreType.DMA,
            pltpu.SemaphoreType.DMA,
        ],
        compiler_params=_SC_PARAMS,
    )(_sort_body)
    return f(em, atom_types, x)


BPS = 8
SM = TM * BPS
NG = CAP // SM


def _gemm_body(bexp_ref, xs_ref, w_ref, b_ref, y1_ref, y2_ref):
    gidx = pl.program_id(0)
    for k in range(BPS):
        e = bexp_ref[gidx * BPS + k]
        xb = xs_ref[pl.ds(k * TM, TM), :].astype(jnp.bfloat16)
        w = w_ref[pl.ds(e, 1), :, :][0]
        y = jnp.tanh(
            jnp.dot(xb, w, preferred_element_type=jnp.float32)
            + b_ref[pl.ds(e, 1), :])
        y1_ref[pl.ds(k * TM, TM), :] = y[:, :HALF_OUT]
        y2_ref[pl.ds(k * TM, TM), :] = y[:, HALF_OUT:]


def _gemm(bexp, xs, expert_W_bf16, expert_b):
    grid_spec = pltpu.PrefetchScalarGridSpec(
        num_scalar_prefetch=1,
        grid=(NG,),
        in_specs=[
            pl.BlockSpec((SM, NUM_IN), lambda b, s: (b, 0)),
            pl.BlockSpec((E, NUM_IN, TOTAL_OUT), lambda b, s: (0, 0, 0)),
            pl.BlockSpec((E, TOTAL_OUT), lambda b, s: (0, 0)),
        ],
        out_specs=[
            pl.BlockSpec((SM, HALF_OUT), lambda b, s: (b, 0)),
            pl.BlockSpec((SM, HALF_OUT), lambda b, s: (b, 0)),
        ],
    )
    return pl.pallas_call(
        _gemm_body,
        grid_spec=grid_spec,
        out_shape=[jax.ShapeDtypeStruct((CAP, HALF_OUT), jnp.float32),
                   jax.ShapeDtypeStruct((CAP, HALF_OUT), jnp.float32)],
    )(bexp, xs, expert_W_bf16, expert_b)


def _gath_body(pos_hbm, y1_hbm, y2_hbm, o1_hbm, o2_hbm,
               pv_v, y1_v, y2_v, g1sem, g2sem, wsem):
    wid = _wid()
    pltpu.sync_copy(pos_hbm.at[pl.ds(wid * RPT, RPT)], pv_v)
    g1 = [pltpu.async_copy(y1_hbm.at[pv_v.at[j]],
                           y1_v.at[pl.ds(j * 128, 128)], g1sem)
          for j in range(RPT)]
    g2 = [pltpu.async_copy(y2_hbm.at[pv_v.at[j]],
                           y2_v.at[pl.ds(j * 128, 128)], g2sem)
          for j in range(RPT)]
    for c in g1:
        c.wait()
    w1 = pltpu.async_copy(y1_v, o1_hbm.at[pl.ds(wid * TPT, TPT)], wsem)
    for c in g2:
        c.wait()
    w2 = pltpu.async_copy(y2_v, o2_hbm.at[pl.ds(wid * TPT, TPT)], wsem)
    w1.wait()
    w2.wait()


def _gather_y(pos, y1, y2):
    f = functools.partial(
        pl.kernel,
        out_type=[jax.ShapeDtypeStruct((N_TOK, HALF_OUT), jnp.float32),
                  jax.ShapeDtypeStruct((N_TOK, HALF_OUT), jnp.float32)],
        mesh=_mesh(),
        scratch_types=[
            pltpu.VMEM((RPT, 128), jnp.int32),
            pltpu.VMEM((TPT, HALF_OUT), jnp.float32),
            pltpu.VMEM((TPT, HALF_OUT), jnp.float32),
            pltpu.SemaphoreType.DMA,
            pltpu.SemaphoreType.DMA,
            pltpu.SemaphoreType.DMA,
        ],
        compiler_params=_SC_PARAMS,
    )(_gath_body)
    return f(pos, y1, y2)


def kernel(x, type_embeddings, atom_types, gate_W, gate_b, expert_W, expert_b):
    atom_types = atom_types.astype(jnp.int32)
    em = _expert_map(type_embeddings, gate_W, gate_b)
    pos, bexp, xs = _sort_scatter(em, atom_types, x)
    y1, y2 = _gemm(bexp, xs, expert_W.astype(jnp.bfloat16), expert_b)
    o1, o2 = _gather_y(pos, y1, y2)
    return (o1, o2)


# speedup vs baseline: 1.7338x; 1.0325x over previous
import functools

import jax
import jax.numpy as jnp
from jax import lax
from jax.experimental import pallas as pl
from jax.experimental.pallas import tpu as pltpu
from jax.experimental.pallas import tpu_sc as plsc

N_TOK = 8192
NUM_IN = 256
TOTAL_OUT = 256
HALF_OUT = 128
E = 16
NTYPES = 128
TEBD = 64

TM = 128
CAP = 10240
NB = CAP // TM
TM_SHIFT = TM.bit_length() - 1

NC, NS = 2, 16
NW = NC * NS
TPT = N_TOK // NW
RPT = TPT // 128
NV = N_TOK // 16


def _wid():
    return lax.axis_index("s") * NC + lax.axis_index("c")


def _mesh():
    return plsc.VectorSubcoreMesh(core_axis_name="c", subcore_axis_name="s")


_SC_PARAMS = pltpu.CompilerParams(needs_layout_passes=False)


def _emap_body(te_ref, gw_ref, gb_ref, em_ref):
    logits = jnp.dot(te_ref[...], gw_ref[...],
                     preferred_element_type=jnp.float32) + gb_ref[...]
    em_ref[...] = jnp.argmax(logits, axis=1).astype(jnp.int32)[None, :]


def _expert_map(type_embeddings, gate_W, gate_b):
    return pl.pallas_call(
        _emap_body,
        out_shape=jax.ShapeDtypeStruct((1, NTYPES), jnp.int32),
    )(type_embeddings, gate_W, gate_b.reshape(1, E))


def _sort_body(em_hbm, at_hbm, x_hbm, pos_hbm, bexp_hbm, xs_hbm,
               em_v, at_v, ev_v, hist_v, eacc_v, ctr_v, pos2_v, bexp_v, xv_v,
               sem, xsem):
    wid = _wid()
    lane = lax.broadcasted_iota(jnp.int32, (E,), 0)
    ones = jnp.ones((E,), jnp.int32)

    xload = pltpu.async_copy(x_hbm.at[pl.ds(wid * TPT, TPT)], xv_v, xsem)
    pltpu.sync_copy(em_hbm.at[0], em_v)
    pltpu.sync_copy(at_hbm, at_v)

    def _zero(i, carry):
        hist_v[pl.ds(i * 16, 16)] = jnp.zeros((E,), jnp.int32)
        return carry
    lax.fori_loop(0, 8 * NTYPES // 16, _zero, 0)

    def _acc8(i, carry):
        for r in range(8):
            tv = at_v[pl.ds((i * 8 + r) * 16, 16)]
            plsc.addupdate_scatter(hist_v, [r * NTYPES + tv], ones)
        return carry

    def _esum():
        eacc_v[...] = jnp.zeros((E,), jnp.int32)
        for tb in range(NTYPES // 16):
            ts = jnp.zeros((E,), jnp.int32)
            for r in range(8):
                ts = ts + hist_v[pl.ds(r * NTYPES + tb * 16, 16)]
            emv = em_v[pl.ds(tb * 16, 16)]
            plsc.addupdate_scatter(eacc_v, [emv], ts)
        return eacc_v[...]

    lax.fori_loop(0, wid * (TPT // 128), _acc8, 0)
    bef = _esum()
    base = wid * (TPT // 16)

    def _acc_own(i, carry):
        tv = at_v[pl.ds((base + i) * 16, 16)]
        evv = plsc.load_gather(em_v, [tv])
        ev_v[pl.ds(i * 16, 16)] = evv
        plsc.addupdate_scatter(hist_v, [(i & 7) * NTYPES + tv], ones)
        return carry
    lax.fori_loop(0, TPT // 16, _acc_own, 0)
    lax.fori_loop((base + TPT // 16) // 8, NV // 8, _acc8, 0)
    g = _esum()

    padded = ((g + (TM - 1)) >> TM_SHIFT) << TM_SHIFT
    offs = plsc.cumsum(padded) - padded
    start = offs + bef

    @pl.when(wid == 0)
    def _():
        for k in range(NB // 16):
            bv = (lane + 16 * k) * TM
            be = jnp.zeros((E,), jnp.int32)
            for e in range(E):
                be = jnp.where((bv >= offs[e]) & (bv < offs[e] + padded[e]),
                               e, be)
            bexp_v[pl.ds(16 * k, 16)] = be
        pltpu.sync_copy(bexp_v, bexp_hbm)

    ctr_v[...] = start

    def _rank(i, carry):
        evv = ev_v[pl.ds(i * 16, 16)]
        base_ = plsc.load_gather(ctr_v, [evv])
        rk, _ = plsc.scan_count(evv)
        posv = jnp.minimum(base_ + rk - 1, CAP - 1)
        li = i * 16 + lane
        plsc.store_scatter(pos2_v, [li >> 7, li & 127], posv)
        plsc.addupdate_scatter(ctr_v, [evv], ones)
        return carry

    cps = []
    for j in range(RPT):
        lax.fori_loop(j * 8, (j + 1) * 8, _rank, 0)
        if j == 0:
            xload.wait()
        cps.append(pltpu.async_copy(xv_v.at[pl.ds(j * 128, 128)],
                                    xs_hbm.at[pos2_v.at[j]], sem))
    cps.append(pltpu.async_copy(pos2_v, pos_hbm.at[pl.ds(wid * RPT, RPT)],
                                xsem))
    for c in cps:
        c.wait()


def _sort_scatter(em, atom_types, x):
    f = functools.partial(
        pl.kernel,
        out_type=[jax.ShapeDtypeStruct((N_TOK // 128, 128), jnp.int32),
                  jax.ShapeDtypeStruct((NB,), jnp.int32),
                  jax.ShapeDtypeStruct((CAP, NUM_IN), jnp.float32)],
        mesh=_mesh(),
        scratch_types=[
            pltpu.VMEM((NTYPES,), jnp.int32),
            pltpu.VMEM((N_TOK,), jnp.int32),
            pltpu.VMEM((TPT,), jnp.int32),
            pltpu.VMEM((8 * NTYPES,), jnp.int32),
            pltpu.VMEM((E,), jnp.int32),
            pltpu.VMEM((E,), jnp.int32),
            pltpu.VMEM((RPT, 128), jnp.int32),
            pltpu.VMEM((NB,), jnp.int32),
            pltpu.VMEM((TPT, NUM_IN), jnp.float32),
            pltpu.SemaphoreType.DMA,
            pltpu.SemaphoreType.DMA,
        ],
        compiler_params=_SC_PARAMS,
    )(_sort_body)
    return f(em, atom_types, x)


BPS = 8
SM = TM * BPS
NG = CAP // SM


def _gemm_body(bexp_ref, xs_ref, w_ref, b_ref, y1_ref, y2_ref):
    gidx = pl.program_id(0)
    for k in range(BPS):
        e = bexp_ref[gidx * BPS + k]
        xb = xs_ref[pl.ds(k * TM, TM), :].astype(jnp.bfloat16)
        w = w_ref[pl.ds(e, 1), :, :][0]
        y = jnp.tanh(
            jnp.dot(xb, w, preferred_element_type=jnp.float32)
            + b_ref[pl.ds(e, 1), :])
        y1_ref[pl.ds(k * TM, TM), :] = y[:, :HALF_OUT]
        y2_ref[pl.ds(k * TM, TM), :] = y[:, HALF_OUT:]


def _gemm(bexp, xs, expert_W_bf16, expert_b):
    grid_spec = pltpu.PrefetchScalarGridSpec(
        num_scalar_prefetch=1,
        grid=(NG,),
        in_specs=[
            pl.BlockSpec((SM, NUM_IN), lambda b, s: (b, 0)),
            pl.BlockSpec((E, NUM_IN, TOTAL_OUT), lambda b, s: (0, 0, 0)),
            pl.BlockSpec((E, TOTAL_OUT), lambda b, s: (0, 0)),
        ],
        out_specs=[
            pl.BlockSpec((SM, HALF_OUT), lambda b, s: (b, 0)),
            pl.BlockSpec((SM, HALF_OUT), lambda b, s: (b, 0)),
        ],
    )
    return pl.pallas_call(
        _gemm_body,
        grid_spec=grid_spec,
        out_shape=[jax.ShapeDtypeStruct((CAP, HALF_OUT), jnp.float32),
                   jax.ShapeDtypeStruct((CAP, HALF_OUT), jnp.float32)],
    )(bexp, xs, expert_W_bf16, expert_b)


def _gath_body(pos_hbm, y1_hbm, y2_hbm, o1_hbm, o2_hbm,
               pv_v, y1_v, y2_v, g1sem, g2sem, wsem):
    wid = _wid()
    pltpu.sync_copy(pos_hbm.at[pl.ds(wid * RPT, RPT)], pv_v)
    g1 = [pltpu.async_copy(y1_hbm.at[pv_v.at[j]],
                           y1_v.at[pl.ds(j * 128, 128)], g1sem)
          for j in range(RPT)]
    g2 = [pltpu.async_copy(y2_hbm.at[pv_v.at[j]],
                           y2_v.at[pl.ds(j * 128, 128)], g2sem)
          for j in range(RPT)]
    for c in g1:
        c.wait()
    w1 = pltpu.async_copy(y1_v, o1_hbm.at[pl.ds(wid * TPT, TPT)], wsem)
    for c in g2:
        c.wait()
    w2 = pltpu.async_copy(y2_v, o2_hbm.at[pl.ds(wid * TPT, TPT)], wsem)
    w1.wait()
    w2.wait()


def _gather_y(pos, y1, y2):
    f = functools.partial(
        pl.kernel,
        out_type=[jax.ShapeDtypeStruct((N_TOK, HALF_OUT), jnp.float32),
                  jax.ShapeDtypeStruct((N_TOK, HALF_OUT), jnp.float32)],
        mesh=_mesh(),
        scratch_types=[
            pltpu.VMEM((RPT, 128), jnp.int32),
            pltpu.VMEM((TPT, HALF_OUT), jnp.float32),
            pltpu.VMEM((TPT, HALF_OUT), jnp.float32),
            pltpu.SemaphoreType.DMA,
            pltpu.SemaphoreType.DMA,
            pltpu.SemaphoreType.DMA,
        ],
        compiler_params=_SC_PARAMS,
    )(_gath_body)
    return f(pos, y1, y2)


def kernel(x, type_embeddings, atom_types, gate_W, gate_b, expert_W, expert_b):
    atom_types = atom_types.astype(jnp.int32)
    em = _expert_map(type_embeddings, gate_W, gate_b)
    pos, bexp, xs = _sort_scatter(em, atom_types, x)
    y1, y2 = _gemm(bexp, xs, expert_W.astype(jnp.bfloat16), expert_b)
    o1, o2 = _gather_y(pos, y1, y2)
    return (o1, o2)


# speedup vs baseline: 1.7752x; 1.0239x over previous
import functools

import jax
import jax.numpy as jnp
from jax import lax
from jax.experimental import pallas as pl
from jax.experimental.pallas import tpu as pltpu
from jax.experimental.pallas import tpu_sc as plsc

N_TOK = 8192
NUM_IN = 256
TOTAL_OUT = 256
HALF_OUT = 128
E = 16
NTYPES = 128
TEBD = 64

TM = 128
CAP = 10240
NB = CAP // TM
TM_SHIFT = TM.bit_length() - 1

NC, NS = 2, 16
NW = NC * NS
TPT = N_TOK // NW
RPT = TPT // 128
NV = N_TOK // 16


def _wid():
    return lax.axis_index("s") * NC + lax.axis_index("c")


def _mesh():
    return plsc.VectorSubcoreMesh(core_axis_name="c", subcore_axis_name="s")


_SC_PARAMS = pltpu.CompilerParams(needs_layout_passes=False)


def _emap_body(te_ref, gw_ref, gb_ref, em_ref):
    logits = jnp.dot(te_ref[...], gw_ref[...],
                     preferred_element_type=jnp.float32) + gb_ref[...]
    em_ref[...] = jnp.argmax(logits, axis=1).astype(jnp.int32)[None, :]


def _expert_map(type_embeddings, gate_W, gate_b):
    return pl.pallas_call(
        _emap_body,
        out_shape=jax.ShapeDtypeStruct((1, NTYPES), jnp.int32),
    )(type_embeddings, gate_W, gate_b.reshape(1, E))


def _sort_body(em_hbm, at_hbm, x_hbm, pos_hbm, bexp_hbm, xs_hbm,
               em_v, at_v, ev_v, hist_v, eacc_v, ctr_v, pos2_v, bexp_v, xv_v,
               sem, xsem):
    wid = _wid()
    lane = lax.broadcasted_iota(jnp.int32, (E,), 0)
    ones = jnp.ones((E,), jnp.int32)

    xload = pltpu.async_copy(x_hbm.at[pl.ds(wid * TPT, TPT)], xv_v, xsem)
    pltpu.sync_copy(em_hbm.at[0], em_v)
    pltpu.sync_copy(at_hbm, at_v)

    def _zero(i, carry):
        hist_v[pl.ds(i * 16, 16)] = jnp.zeros((E,), jnp.int32)
        return carry
    lax.fori_loop(0, 8 * NTYPES // 16, _zero, 0)

    def _acc8(i, carry):
        for r in range(8):
            tv = at_v[pl.ds((i * 8 + r) * 16, 16)]
            plsc.addupdate_scatter(hist_v, [r * NTYPES + tv], ones)
        return carry

    def _esum():
        eacc_v[...] = jnp.zeros((E,), jnp.int32)
        for tb in range(NTYPES // 16):
            ts = jnp.zeros((E,), jnp.int32)
            for r in range(8):
                ts = ts + hist_v[pl.ds(r * NTYPES + tb * 16, 16)]
            emv = em_v[pl.ds(tb * 16, 16)]
            plsc.addupdate_scatter(eacc_v, [emv], ts)
        return eacc_v[...]

    lax.fori_loop(0, wid * (TPT // 128), _acc8, 0)
    bef = _esum()
    base = wid * (TPT // 16)

    def _acc_own(i, carry):
        tv = at_v[pl.ds((base + i) * 16, 16)]
        evv = plsc.load_gather(em_v, [tv])
        ev_v[pl.ds(i * 16, 16)] = evv
        plsc.addupdate_scatter(hist_v, [(i & 7) * NTYPES + tv], ones)
        return carry
    lax.fori_loop(0, TPT // 16, _acc_own, 0)
    lax.fori_loop((base + TPT // 16) // 8, NV // 8, _acc8, 0)
    g = _esum()

    padded = ((g + (TM - 1)) >> TM_SHIFT) << TM_SHIFT
    offs = plsc.cumsum(padded) - padded
    start = offs + bef

    @pl.when(wid == 0)
    def _():
        for k in range(NB // 16):
            bv = (lane + 16 * k) * TM
            be = jnp.zeros((E,), jnp.int32)
            for e in range(E):
                be = jnp.where((bv >= offs[e]) & (bv < offs[e] + padded[e]),
                               e, be)
            bexp_v[pl.ds(16 * k, 16)] = be
        pltpu.sync_copy(bexp_v, bexp_hbm)

    ctr_v[...] = start

    def _rank(i, carry):
        evv = ev_v[pl.ds(i * 16, 16)]
        base_ = plsc.load_gather(ctr_v, [evv])
        rk, _ = plsc.scan_count(evv)
        posv = jnp.minimum(base_ + rk - 1, CAP - 1)
        li = i * 16 + lane
        plsc.store_scatter(pos2_v, [li >> 7, li & 127], posv)
        plsc.addupdate_scatter(ctr_v, [evv], ones)
        return carry

    cps = []
    for j in range(RPT):
        lax.fori_loop(j * 8, (j + 1) * 8, _rank, 0)
        if j == 0:
            xload.wait()
        cps.append(pltpu.async_copy(xv_v.at[pl.ds(j * 128, 128)],
                                    xs_hbm.at[pos2_v.at[j]], sem))
    cps.append(pltpu.async_copy(pos2_v, pos_hbm.at[pl.ds(wid * RPT, RPT)],
                                xsem))
    for c in cps:
        c.wait()


def _sort_scatter(em, atom_types, x):
    f = functools.partial(
        pl.kernel,
        out_type=[jax.ShapeDtypeStruct((N_TOK // 128, 128), jnp.int32),
                  jax.ShapeDtypeStruct((NB,), jnp.int32),
                  jax.ShapeDtypeStruct((CAP, NUM_IN), jnp.float32)],
        mesh=_mesh(),
        scratch_types=[
            pltpu.VMEM((NTYPES,), jnp.int32),
            pltpu.VMEM((N_TOK,), jnp.int32),
            pltpu.VMEM((TPT,), jnp.int32),
            pltpu.VMEM((8 * NTYPES,), jnp.int32),
            pltpu.VMEM((E,), jnp.int32),
            pltpu.VMEM((E,), jnp.int32),
            pltpu.VMEM((RPT, 128), jnp.int32),
            pltpu.VMEM((NB,), jnp.int32),
            pltpu.VMEM((TPT, NUM_IN), jnp.float32),
            pltpu.SemaphoreType.DMA,
            pltpu.SemaphoreType.DMA,
        ],
        compiler_params=_SC_PARAMS,
    )(_sort_body)
    return f(em, atom_types, x)


BPS = 10
SM = TM * BPS
NG = CAP // SM


def _gemm_body(bexp_ref, xs_ref, w_ref, b_ref, y1_ref, y2_ref):
    gidx = pl.program_id(0)
    for k in range(BPS):
        e = bexp_ref[gidx * BPS + k]
        xb = xs_ref[pl.ds(k * TM, TM), :].astype(jnp.bfloat16)
        w = w_ref[pl.ds(e, 1), :, :][0]
        y = jnp.tanh(
            jnp.dot(xb, w, preferred_element_type=jnp.float32)
            + b_ref[pl.ds(e, 1), :])
        y1_ref[pl.ds(k * TM, TM), :] = y[:, :HALF_OUT]
        y2_ref[pl.ds(k * TM, TM), :] = y[:, HALF_OUT:]


def _gemm(bexp, xs, expert_W_bf16, expert_b):
    grid_spec = pltpu.PrefetchScalarGridSpec(
        num_scalar_prefetch=1,
        grid=(NG,),
        in_specs=[
            pl.BlockSpec((SM, NUM_IN), lambda b, s: (b, 0)),
            pl.BlockSpec((E, NUM_IN, TOTAL_OUT), lambda b, s: (0, 0, 0)),
            pl.BlockSpec((E, TOTAL_OUT), lambda b, s: (0, 0)),
        ],
        out_specs=[
            pl.BlockSpec((SM, HALF_OUT), lambda b, s: (b, 0)),
            pl.BlockSpec((SM, HALF_OUT), lambda b, s: (b, 0)),
        ],
    )
    return pl.pallas_call(
        _gemm_body,
        grid_spec=grid_spec,
        out_shape=[jax.ShapeDtypeStruct((CAP, HALF_OUT), jnp.float32),
                   jax.ShapeDtypeStruct((CAP, HALF_OUT), jnp.float32)],
    )(bexp, xs, expert_W_bf16, expert_b)


def _gath_body(pos_hbm, y1_hbm, y2_hbm, o1_hbm, o2_hbm,
               pv_v, y1_v, y2_v, g1sem, g2sem, wsem):
    wid = _wid()
    pltpu.sync_copy(pos_hbm.at[pl.ds(wid * RPT, RPT)], pv_v)
    g1 = [pltpu.async_copy(y1_hbm.at[pv_v.at[j]],
                           y1_v.at[pl.ds(j * 128, 128)], g1sem)
          for j in range(RPT)]
    g2 = [pltpu.async_copy(y2_hbm.at[pv_v.at[j]],
                           y2_v.at[pl.ds(j * 128, 128)], g2sem)
          for j in range(RPT)]
    for c in g1:
        c.wait()
    w1 = pltpu.async_copy(y1_v, o1_hbm.at[pl.ds(wid * TPT, TPT)], wsem)
    for c in g2:
        c.wait()
    w2 = pltpu.async_copy(y2_v, o2_hbm.at[pl.ds(wid * TPT, TPT)], wsem)
    w1.wait()
    w2.wait()


def _gather_y(pos, y1, y2):
    f = functools.partial(
        pl.kernel,
        out_type=[jax.ShapeDtypeStruct((N_TOK, HALF_OUT), jnp.float32),
                  jax.ShapeDtypeStruct((N_TOK, HALF_OUT), jnp.float32)],
        mesh=_mesh(),
        scratch_types=[
            pltpu.VMEM((RPT, 128), jnp.int32),
            pltpu.VMEM((TPT, HALF_OUT), jnp.float32),
            pltpu.VMEM((TPT, HALF_OUT), jnp.float32),
            pltpu.SemaphoreType.DMA,
            pltpu.SemaphoreType.DMA,
            pltpu.SemaphoreType.DMA,
        ],
        compiler_params=_SC_PARAMS,
    )(_gath_body)
    return f(pos, y1, y2)


def kernel(x, type_embeddings, atom_types, gate_W, gate_b, expert_W, expert_b):
    atom_types = atom_types.astype(jnp.int32)
    em = _expert_map(type_embeddings, gate_W, gate_b)
    pos, bexp, xs = _sort_scatter(em, atom_types, x)
    y1, y2 = _gemm(bexp, xs, expert_W.astype(jnp.bfloat16), expert_b)
    o1, o2 = _gather_y(pos, y1, y2)
    return (o1, o2)


# speedup vs baseline: 1.8061x; 1.0174x over previous
import functools

import jax
import jax.numpy as jnp
from jax import lax
from jax.experimental import pallas as pl
from jax.experimental.pallas import tpu as pltpu
from jax.experimental.pallas import tpu_sc as plsc

N_TOK = 8192
NUM_IN = 256
TOTAL_OUT = 256
HALF_OUT = 128
E = 16
NTYPES = 128
TEBD = 64

TM = 128
CAP = 10240
NB = CAP // TM
TM_SHIFT = TM.bit_length() - 1

NC, NS = 2, 16
NW = NC * NS
TPT = N_TOK // NW
RPT = TPT // 128
NV = N_TOK // 16


def _wid():
    return lax.axis_index("s") * NC + lax.axis_index("c")


def _mesh():
    return plsc.VectorSubcoreMesh(core_axis_name="c", subcore_axis_name="s")


_SC_PARAMS = pltpu.CompilerParams(needs_layout_passes=False)


def _emap_body(te_ref, gw_ref, gb_ref, em_ref):
    logits = jnp.dot(te_ref[...], gw_ref[...],
                     preferred_element_type=jnp.float32) + gb_ref[...]
    em_ref[...] = jnp.argmax(logits, axis=1).astype(jnp.int32)[None, :]


def _expert_map(type_embeddings, gate_W, gate_b):
    return pl.pallas_call(
        _emap_body,
        out_shape=jax.ShapeDtypeStruct((1, NTYPES), jnp.int32),
    )(type_embeddings, gate_W, gate_b.reshape(1, E))


def _sort_body(em_hbm, at_hbm, x_hbm, pos_hbm, bexp_hbm, xs_hbm,
               em_v, at_v, ev_v, hist_v, eacc_v, ctr_v, pos2_v, bexp_v, xv_v,
               sem, xsem):
    wid = _wid()
    lane = lax.broadcasted_iota(jnp.int32, (E,), 0)
    ones = jnp.ones((E,), jnp.int32)

    xload = pltpu.async_copy(x_hbm.at[pl.ds(wid * TPT, TPT)], xv_v, xsem)
    pltpu.sync_copy(em_hbm.at[0], em_v)
    pltpu.sync_copy(at_hbm, at_v)

    def _zero(i, carry):
        hist_v[pl.ds(i * 16, 16)] = jnp.zeros((E,), jnp.int32)
        return carry
    lax.fori_loop(0, 8 * NTYPES // 16, _zero, 0)

    def _acc8(i, carry):
        for r in range(8):
            tv = at_v[pl.ds((i * 8 + r) * 16, 16)]
            plsc.addupdate_scatter(hist_v, [r * NTYPES + tv], ones)
        return carry

    def _esum():
        eacc_v[...] = jnp.zeros((E,), jnp.int32)
        for tb in range(NTYPES // 16):
            ts = jnp.zeros((E,), jnp.int32)
            for r in range(8):
                ts = ts + hist_v[pl.ds(r * NTYPES + tb * 16, 16)]
            emv = em_v[pl.ds(tb * 16, 16)]
            plsc.addupdate_scatter(eacc_v, [emv], ts)
        return eacc_v[...]

    lax.fori_loop(0, wid * (TPT // 128), _acc8, 0)
    bef = _esum()
    base = wid * (TPT // 16)

    def _acc_own(i, carry):
        tv = at_v[pl.ds((base + i) * 16, 16)]
        evv = plsc.load_gather(em_v, [tv])
        ev_v[pl.ds(i * 16, 16)] = evv
        plsc.addupdate_scatter(hist_v, [(i & 7) * NTYPES + tv], ones)
        return carry
    lax.fori_loop(0, TPT // 16, _acc_own, 0)
    lax.fori_loop((base + TPT // 16) // 8, NV // 8, _acc8, 0)
    g = _esum()

    padded = ((g + (TM - 1)) >> TM_SHIFT) << TM_SHIFT
    offs = plsc.cumsum(padded) - padded
    start = offs + bef

    @pl.when(wid == 0)
    def _():
        for k in range(NB // 16):
            bv = (lane + 16 * k) * TM
            be = jnp.zeros((E,), jnp.int32)
            for e in range(E):
                be = jnp.where((bv >= offs[e]) & (bv < offs[e] + padded[e]),
                               e, be)
            bexp_v[pl.ds(16 * k, 16)] = be
        pltpu.sync_copy(bexp_v, bexp_hbm)

    ctr_v[...] = start

    def _rank(i, carry):
        evv = ev_v[pl.ds(i * 16, 16)]
        base_ = plsc.load_gather(ctr_v, [evv])
        rk, _ = plsc.scan_count(evv)
        posv = jnp.minimum(base_ + rk - 1, CAP - 1)
        li = i * 16 + lane
        plsc.store_scatter(pos2_v, [li >> 7, li & 127], posv)
        plsc.addupdate_scatter(ctr_v, [evv], ones)
        return carry

    cps = []
    for j in range(RPT):
        lax.fori_loop(j * 8, (j + 1) * 8, _rank, 0)
        if j == 0:
            xload.wait()
        cps.append(pltpu.async_copy(xv_v.at[pl.ds(j * 128, 128)],
                                    xs_hbm.at[pos2_v.at[j]], sem))
    cps.append(pltpu.async_copy(pos2_v, pos_hbm.at[pl.ds(wid * RPT, RPT)],
                                xsem))
    for c in cps:
        c.wait()


def _sort_scatter(em, atom_types, x):
    f = functools.partial(
        pl.kernel,
        out_type=[jax.ShapeDtypeStruct((N_TOK // 128, 128), jnp.int32),
                  jax.ShapeDtypeStruct((NB,), jnp.int32),
                  jax.ShapeDtypeStruct((CAP, NUM_IN), jnp.float32)],
        mesh=_mesh(),
        scratch_types=[
            pltpu.VMEM((NTYPES,), jnp.int32),
            pltpu.VMEM((N_TOK,), jnp.int32),
            pltpu.VMEM((TPT,), jnp.int32),
            pltpu.VMEM((8 * NTYPES,), jnp.int32),
            pltpu.VMEM((E,), jnp.int32),
            pltpu.VMEM((E,), jnp.int32),
            pltpu.VMEM((RPT, 128), jnp.int32),
            pltpu.VMEM((NB,), jnp.int32),
            pltpu.VMEM((TPT, NUM_IN), jnp.float32),
            pltpu.SemaphoreType.DMA,
            pltpu.SemaphoreType.DMA,
        ],
        compiler_params=_SC_PARAMS,
    )(_sort_body)
    return f(em, atom_types, x)


BPS = 16
SM = TM * BPS
NG = CAP // SM


def _gemm_body(bexp_ref, xs_ref, w_ref, b_ref, y1_ref, y2_ref):
    gidx = pl.program_id(0)
    for k in range(BPS):
        e = bexp_ref[gidx * BPS + k]
        xb = xs_ref[pl.ds(k * TM, TM), :].astype(jnp.bfloat16)
        w = w_ref[pl.ds(e, 1), :, :][0]
        y = jnp.tanh(
            jnp.dot(xb, w, preferred_element_type=jnp.float32)
            + b_ref[pl.ds(e, 1), :])
        y1_ref[pl.ds(k * TM, TM), :] = y[:, :HALF_OUT]
        y2_ref[pl.ds(k * TM, TM), :] = y[:, HALF_OUT:]


def _gemm(bexp, xs, expert_W_bf16, expert_b):
    grid_spec = pltpu.PrefetchScalarGridSpec(
        num_scalar_prefetch=1,
        grid=(NG,),
        in_specs=[
            pl.BlockSpec((SM, NUM_IN), lambda b, s: (b, 0)),
            pl.BlockSpec((E, NUM_IN, TOTAL_OUT), lambda b, s: (0, 0, 0)),
            pl.BlockSpec((E, TOTAL_OUT), lambda b, s: (0, 0)),
        ],
        out_specs=[
            pl.BlockSpec((SM, HALF_OUT), lambda b, s: (b, 0)),
            pl.BlockSpec((SM, HALF_OUT), lambda b, s: (b, 0)),
        ],
    )
    return pl.pallas_call(
        _gemm_body,
        grid_spec=grid_spec,
        out_shape=[jax.ShapeDtypeStruct((CAP, HALF_OUT), jnp.float32),
                   jax.ShapeDtypeStruct((CAP, HALF_OUT), jnp.float32)],
    )(bexp, xs, expert_W_bf16, expert_b)


def _gath_body(pos_hbm, y1_hbm, y2_hbm, o1_hbm, o2_hbm,
               pv_v, y1_v, y2_v, g1sem, g2sem, wsem):
    wid = _wid()
    pltpu.sync_copy(pos_hbm.at[pl.ds(wid * RPT, RPT)], pv_v)
    g1 = [pltpu.async_copy(y1_hbm.at[pv_v.at[j]],
                           y1_v.at[pl.ds(j * 128, 128)], g1sem)
          for j in range(RPT)]
    g2 = [pltpu.async_copy(y2_hbm.at[pv_v.at[j]],
                           y2_v.at[pl.ds(j * 128, 128)], g2sem)
          for j in range(RPT)]
    for c in g1:
        c.wait()
    w1 = pltpu.async_copy(y1_v, o1_hbm.at[pl.ds(wid * TPT, TPT)], wsem)
    for c in g2:
        c.wait()
    w2 = pltpu.async_copy(y2_v, o2_hbm.at[pl.ds(wid * TPT, TPT)], wsem)
    w1.wait()
    w2.wait()


def _gather_y(pos, y1, y2):
    f = functools.partial(
        pl.kernel,
        out_type=[jax.ShapeDtypeStruct((N_TOK, HALF_OUT), jnp.float32),
                  jax.ShapeDtypeStruct((N_TOK, HALF_OUT), jnp.float32)],
        mesh=_mesh(),
        scratch_types=[
            pltpu.VMEM((RPT, 128), jnp.int32),
            pltpu.VMEM((TPT, HALF_OUT), jnp.float32),
            pltpu.VMEM((TPT, HALF_OUT), jnp.float32),
            pltpu.SemaphoreType.DMA,
            pltpu.SemaphoreType.DMA,
            pltpu.SemaphoreType.DMA,
        ],
        compiler_params=_SC_PARAMS,
    )(_gath_body)
    return f(pos, y1, y2)


def kernel(x, type_embeddings, atom_types, gate_W, gate_b, expert_W, expert_b):
    atom_types = atom_types.astype(jnp.int32)
    em = _expert_map(type_embeddings, gate_W, gate_b)
    pos, bexp, xs = _sort_scatter(em, atom_types, x)
    y1, y2 = _gemm(bexp, xs, expert_W.astype(jnp.bfloat16), expert_b)
    o1, o2 = _gather_y(pos, y1, y2)
    return (o1, o2)


# speedup vs baseline: 1.8565x; 1.0279x over previous
import functools

import jax
import jax.numpy as jnp
from jax import lax
from jax.experimental import pallas as pl
from jax.experimental.pallas import tpu as pltpu
from jax.experimental.pallas import tpu_sc as plsc

N_TOK = 8192
NUM_IN = 256
TOTAL_OUT = 256
HALF_OUT = 128
E = 16
NTYPES = 128
TEBD = 64

TM = 128
CAP = 10240
NB = CAP // TM
TM_SHIFT = TM.bit_length() - 1

NC, NS = 2, 16
NW = NC * NS
TPT = N_TOK // NW
RPT = TPT // 128
NV = N_TOK // 16


def _wid():
    return lax.axis_index("s") * NC + lax.axis_index("c")


def _mesh():
    return plsc.VectorSubcoreMesh(core_axis_name="c", subcore_axis_name="s")


_SC_PARAMS = pltpu.CompilerParams(needs_layout_passes=False)


def _emap_body(te_ref, gw_ref, gb_ref, em_ref):
    logits = jnp.dot(te_ref[...], gw_ref[...],
                     preferred_element_type=jnp.float32) + gb_ref[...]
    em_ref[...] = jnp.argmax(logits, axis=1).astype(jnp.int32)[None, :]


def _expert_map(type_embeddings, gate_W, gate_b):
    return pl.pallas_call(
        _emap_body,
        out_shape=jax.ShapeDtypeStruct((1, NTYPES), jnp.int32),
    )(type_embeddings, gate_W, gate_b.reshape(1, E))


def _sort_body(em_hbm, at_hbm, x_hbm, pos_hbm, bexp_hbm, xs_hbm,
               em_v, at_v, ev_v, hist_v, eacc_v, ctr_v, pos2_v, bexp_v, xv_v,
               sem, xsem):
    wid = _wid()
    lane = lax.broadcasted_iota(jnp.int32, (E,), 0)
    ones = jnp.ones((E,), jnp.int32)

    xload = pltpu.async_copy(x_hbm.at[pl.ds(wid * TPT, TPT)], xv_v, xsem)
    pltpu.sync_copy(em_hbm.at[0], em_v)
    pltpu.sync_copy(at_hbm, at_v)

    def _zero(i, carry):
        hist_v[pl.ds(i * 16, 16)] = jnp.zeros((E,), jnp.int32)
        return carry
    lax.fori_loop(0, 8 * NTYPES // 16, _zero, 0)

    def _acc8(i, carry):
        for r in range(8):
            tv = at_v[pl.ds((i * 8 + r) * 16, 16)]
            plsc.addupdate_scatter(hist_v, [r * NTYPES + tv], ones)
        return carry

    def _esum():
        eacc_v[...] = jnp.zeros((E,), jnp.int32)
        for tb in range(NTYPES // 16):
            ts = jnp.zeros((E,), jnp.int32)
            for r in range(8):
                ts = ts + hist_v[pl.ds(r * NTYPES + tb * 16, 16)]
            emv = em_v[pl.ds(tb * 16, 16)]
            plsc.addupdate_scatter(eacc_v, [emv], ts)
        return eacc_v[...]

    lax.fori_loop(0, wid * (TPT // 128), _acc8, 0)
    bef = _esum()
    base = wid * (TPT // 16)

    def _acc_own(i, carry):
        tv = at_v[pl.ds((base + i) * 16, 16)]
        evv = plsc.load_gather(em_v, [tv])
        ev_v[pl.ds(i * 16, 16)] = evv
        plsc.addupdate_scatter(hist_v, [(i & 7) * NTYPES + tv], ones)
        return carry
    lax.fori_loop(0, TPT // 16, _acc_own, 0)
    lax.fori_loop((base + TPT // 16) // 8, NV // 8, _acc8, 0)
    g = _esum()

    padded = ((g + (TM - 1)) >> TM_SHIFT) << TM_SHIFT
    offs = plsc.cumsum(padded) - padded
    start = offs + bef

    @pl.when(wid == 0)
    def _():
        for k in range(NB // 16):
            bv = (lane + 16 * k) * TM
            be = jnp.zeros((E,), jnp.int32)
            for e in range(E):
                be = jnp.where((bv >= offs[e]) & (bv < offs[e] + padded[e]),
                               e, be)
            bexp_v[pl.ds(16 * k, 16)] = be
        pltpu.sync_copy(bexp_v, bexp_hbm)

    ctr_v[...] = start

    def _rank(i, carry):
        evv = ev_v[pl.ds(i * 16, 16)]
        base_ = plsc.load_gather(ctr_v, [evv])
        rk, _ = plsc.scan_count(evv)
        posv = jnp.minimum(base_ + rk - 1, CAP - 1)
        li = i * 16 + lane
        plsc.store_scatter(pos2_v, [li >> 7, li & 127], posv)
        plsc.addupdate_scatter(ctr_v, [evv], ones)
        return carry

    cps = []
    for j in range(RPT):
        lax.fori_loop(j * 8, (j + 1) * 8, _rank, 0)
        if j == 0:
            xload.wait()
        cps.append(pltpu.async_copy(xv_v.at[pl.ds(j * 128, 128)],
                                    xs_hbm.at[pos2_v.at[j]], sem))
    cps.append(pltpu.async_copy(pos2_v, pos_hbm.at[pl.ds(wid * RPT, RPT)],
                                xsem))
    for c in cps:
        c.wait()


def _sort_scatter(em, atom_types, x):
    f = functools.partial(
        pl.kernel,
        out_type=[jax.ShapeDtypeStruct((N_TOK // 128, 128), jnp.int32),
                  jax.ShapeDtypeStruct((NB,), jnp.int32),
                  jax.ShapeDtypeStruct((CAP, NUM_IN), jnp.float32)],
        mesh=_mesh(),
        scratch_types=[
            pltpu.VMEM((NTYPES,), jnp.int32),
            pltpu.VMEM((N_TOK,), jnp.int32),
            pltpu.VMEM((TPT,), jnp.int32),
            pltpu.VMEM((8 * NTYPES,), jnp.int32),
            pltpu.VMEM((E,), jnp.int32),
            pltpu.VMEM((E,), jnp.int32),
            pltpu.VMEM((RPT, 128), jnp.int32),
            pltpu.VMEM((NB,), jnp.int32),
            pltpu.VMEM((TPT, NUM_IN), jnp.float32),
            pltpu.SemaphoreType.DMA,
            pltpu.SemaphoreType.DMA,
        ],
        compiler_params=_SC_PARAMS,
    )(_sort_body)
    return f(em, atom_types, x)


BPS = 20
SM = TM * BPS
NG = CAP // SM


def _gemm_body(bexp_ref, xs_ref, w_ref, b_ref, y1_ref, y2_ref):
    gidx = pl.program_id(0)
    for k in range(BPS):
        e = bexp_ref[gidx * BPS + k]
        xb = xs_ref[pl.ds(k * TM, TM), :].astype(jnp.bfloat16)
        w = w_ref[pl.ds(e, 1), :, :][0]
        y = jnp.tanh(
            jnp.dot(xb, w, preferred_element_type=jnp.float32)
            + b_ref[pl.ds(e, 1), :])
        y1_ref[pl.ds(k * TM, TM), :] = y[:, :HALF_OUT]
        y2_ref[pl.ds(k * TM, TM), :] = y[:, HALF_OUT:]


def _gemm(bexp, xs, expert_W_bf16, expert_b):
    grid_spec = pltpu.PrefetchScalarGridSpec(
        num_scalar_prefetch=1,
        grid=(NG,),
        in_specs=[
            pl.BlockSpec((SM, NUM_IN), lambda b, s: (b, 0)),
            pl.BlockSpec((E, NUM_IN, TOTAL_OUT), lambda b, s: (0, 0, 0)),
            pl.BlockSpec((E, TOTAL_OUT), lambda b, s: (0, 0)),
        ],
        out_specs=[
            pl.BlockSpec((SM, HALF_OUT), lambda b, s: (b, 0)),
            pl.BlockSpec((SM, HALF_OUT), lambda b, s: (b, 0)),
        ],
    )
    return pl.pallas_call(
        _gemm_body,
        grid_spec=grid_spec,
        out_shape=[jax.ShapeDtypeStruct((CAP, HALF_OUT), jnp.float32),
                   jax.ShapeDtypeStruct((CAP, HALF_OUT), jnp.float32)],
    )(bexp, xs, expert_W_bf16, expert_b)


def _gath_body(pos_hbm, y1_hbm, y2_hbm, o1_hbm, o2_hbm,
               pv_v, y1_v, y2_v, g1sem, g2sem, wsem):
    wid = _wid()
    pltpu.sync_copy(pos_hbm.at[pl.ds(wid * RPT, RPT)], pv_v)
    g1 = [pltpu.async_copy(y1_hbm.at[pv_v.at[j]],
                           y1_v.at[pl.ds(j * 128, 128)], g1sem)
          for j in range(RPT)]
    g2 = [pltpu.async_copy(y2_hbm.at[pv_v.at[j]],
                           y2_v.at[pl.ds(j * 128, 128)], g2sem)
          for j in range(RPT)]
    for c in g1:
        c.wait()
    w1 = pltpu.async_copy(y1_v, o1_hbm.at[pl.ds(wid * TPT, TPT)], wsem)
    for c in g2:
        c.wait()
    w2 = pltpu.async_copy(y2_v, o2_hbm.at[pl.ds(wid * TPT, TPT)], wsem)
    w1.wait()
    w2.wait()


def _gather_y(pos, y1, y2):
    f = functools.partial(
        pl.kernel,
        out_type=[jax.ShapeDtypeStruct((N_TOK, HALF_OUT), jnp.float32),
                  jax.ShapeDtypeStruct((N_TOK, HALF_OUT), jnp.float32)],
        mesh=_mesh(),
        scratch_types=[
            pltpu.VMEM((RPT, 128), jnp.int32),
            pltpu.VMEM((TPT, HALF_OUT), jnp.float32),
            pltpu.VMEM((TPT, HALF_OUT), jnp.float32),
            pltpu.SemaphoreType.DMA,
            pltpu.SemaphoreType.DMA,
            pltpu.SemaphoreType.DMA,
        ],
        compiler_params=_SC_PARAMS,
    )(_gath_body)
    return f(pos, y1, y2)


def kernel(x, type_embeddings, atom_types, gate_W, gate_b, expert_W, expert_b):
    atom_types = atom_types.astype(jnp.int32)
    em = _expert_map(type_embeddings, gate_W, gate_b)
    pos, bexp, xs = _sort_scatter(em, atom_types, x)
    y1, y2 = _gemm(bexp, xs, expert_W.astype(jnp.bfloat16), expert_b)
    o1, o2 = _gather_y(pos, y1, y2)
    return (o1, o2)


# speedup vs baseline: 1.8970x; 1.0218x over previous
import functools

import jax
import jax.numpy as jnp
from jax import lax
from jax.experimental import pallas as pl
from jax.experimental.pallas import tpu as pltpu
from jax.experimental.pallas import tpu_sc as plsc

N_TOK = 8192
NUM_IN = 256
TOTAL_OUT = 256
HALF_OUT = 128
E = 16
NTYPES = 128
TEBD = 64

TM = 128
CAP = 10240
NB = CAP // TM
TM_SHIFT = TM.bit_length() - 1

NC, NS = 2, 16
NW = NC * NS
TPT = N_TOK // NW
RPT = TPT // 128
NV = N_TOK // 16


def _wid():
    return lax.axis_index("s") * NC + lax.axis_index("c")


def _mesh():
    return plsc.VectorSubcoreMesh(core_axis_name="c", subcore_axis_name="s")


_SC_PARAMS = pltpu.CompilerParams(needs_layout_passes=False)


def _emap_body(te_ref, gw_ref, gb_ref, em_ref):
    logits = jnp.dot(te_ref[...], gw_ref[...],
                     preferred_element_type=jnp.float32) + gb_ref[...]
    em_ref[...] = jnp.argmax(logits, axis=1).astype(jnp.int32)[None, :]


def _expert_map(type_embeddings, gate_W, gate_b):
    return pl.pallas_call(
        _emap_body,
        out_shape=jax.ShapeDtypeStruct((1, NTYPES), jnp.int32),
    )(type_embeddings, gate_W, gate_b.reshape(1, E))


def _sort_body(em_hbm, at_hbm, x_hbm, pos_hbm, bexp_hbm, xs_hbm,
               em_v, at_v, ev_v, hist_v, eacc_v, ctr_v, pos2_v, bexp_v, xv_v,
               sem, xsem):
    wid = _wid()
    lane = lax.broadcasted_iota(jnp.int32, (E,), 0)
    ones = jnp.ones((E,), jnp.int32)

    xload = pltpu.async_copy(x_hbm.at[pl.ds(wid * TPT, TPT)], xv_v, xsem)
    pltpu.sync_copy(em_hbm.at[0], em_v)
    pltpu.sync_copy(at_hbm, at_v)

    def _zero(i, carry):
        hist_v[pl.ds(i * 16, 16)] = jnp.zeros((E,), jnp.int32)
        return carry
    lax.fori_loop(0, 8 * NTYPES // 16, _zero, 0)

    def _acc8(i, carry):
        for r in range(8):
            tv = at_v[pl.ds((i * 8 + r) * 16, 16)]
            plsc.addupdate_scatter(hist_v, [r * NTYPES + tv], ones)
        return carry

    def _esum():
        eacc_v[...] = jnp.zeros((E,), jnp.int32)
        for tb in range(NTYPES // 16):
            ts = jnp.zeros((E,), jnp.int32)
            for r in range(8):
                ts = ts + hist_v[pl.ds(r * NTYPES + tb * 16, 16)]
            emv = em_v[pl.ds(tb * 16, 16)]
            plsc.addupdate_scatter(eacc_v, [emv], ts)
        return eacc_v[...]

    lax.fori_loop(0, wid * (TPT // 128), _acc8, 0)
    bef = _esum()
    base = wid * (TPT // 16)

    def _acc_own(i, carry):
        tv = at_v[pl.ds((base + i) * 16, 16)]
        evv = plsc.load_gather(em_v, [tv])
        ev_v[pl.ds(i * 16, 16)] = evv
        plsc.addupdate_scatter(hist_v, [(i & 7) * NTYPES + tv], ones)
        return carry
    lax.fori_loop(0, TPT // 16, _acc_own, 0)
    lax.fori_loop((base + TPT // 16) // 8, NV // 8, _acc8, 0)
    g = _esum()

    padded = ((g + (TM - 1)) >> TM_SHIFT) << TM_SHIFT
    offs = plsc.cumsum(padded) - padded
    start = offs + bef

    @pl.when(wid == 0)
    def _():
        for k in range(NB // 16):
            bv = (lane + 16 * k) * TM
            be = jnp.zeros((E,), jnp.int32)
            for e in range(E):
                be = jnp.where((bv >= offs[e]) & (bv < offs[e] + padded[e]),
                               e, be)
            bexp_v[pl.ds(16 * k, 16)] = be
        pltpu.sync_copy(bexp_v, bexp_hbm)

    ctr_v[...] = start

    def _rank(i, carry):
        evv = ev_v[pl.ds(i * 16, 16)]
        base_ = plsc.load_gather(ctr_v, [evv])
        rk, _ = plsc.scan_count(evv)
        posv = jnp.minimum(base_ + rk - 1, CAP - 1)
        li = i * 16 + lane
        plsc.store_scatter(pos2_v, [li >> 7, li & 127], posv)
        plsc.addupdate_scatter(ctr_v, [evv], ones)
        return carry

    cps = []
    for j in range(RPT):
        lax.fori_loop(j * 8, (j + 1) * 8, _rank, 0)
        if j == 0:
            xload.wait()
        cps.append(pltpu.async_copy(xv_v.at[pl.ds(j * 128, 128)],
                                    xs_hbm.at[pos2_v.at[j]], sem))
    cps.append(pltpu.async_copy(pos2_v, pos_hbm.at[pl.ds(wid * RPT, RPT)],
                                xsem))
    for c in cps:
        c.wait()


def _sort_scatter(em, atom_types, x):
    f = functools.partial(
        pl.kernel,
        out_type=[jax.ShapeDtypeStruct((N_TOK // 128, 128), jnp.int32),
                  jax.ShapeDtypeStruct((NB,), jnp.int32),
                  jax.ShapeDtypeStruct((CAP, NUM_IN), jnp.float32)],
        mesh=_mesh(),
        scratch_types=[
            pltpu.VMEM((NTYPES,), jnp.int32),
            pltpu.VMEM((N_TOK,), jnp.int32),
            pltpu.VMEM((TPT,), jnp.int32),
            pltpu.VMEM((8 * NTYPES,), jnp.int32),
            pltpu.VMEM((E,), jnp.int32),
            pltpu.VMEM((E,), jnp.int32),
            pltpu.VMEM((RPT, 128), jnp.int32),
            pltpu.VMEM((NB,), jnp.int32),
            pltpu.VMEM((TPT, NUM_IN), jnp.float32),
            pltpu.SemaphoreType.DMA,
            pltpu.SemaphoreType.DMA,
        ],
        compiler_params=_SC_PARAMS,
    )(_sort_body)
    return f(em, atom_types, x)


BPS = 40
SM = TM * BPS
NG = CAP // SM


def _gemm_body(bexp_ref, xs_ref, w_ref, b_ref, y1_ref, y2_ref):
    gidx = pl.program_id(0)
    for k in range(BPS):
        e = bexp_ref[gidx * BPS + k]
        xb = xs_ref[pl.ds(k * TM, TM), :].astype(jnp.bfloat16)
        w = w_ref[pl.ds(e, 1), :, :][0]
        y = jnp.tanh(
            jnp.dot(xb, w, preferred_element_type=jnp.float32)
            + b_ref[pl.ds(e, 1), :])
        y1_ref[pl.ds(k * TM, TM), :] = y[:, :HALF_OUT]
        y2_ref[pl.ds(k * TM, TM), :] = y[:, HALF_OUT:]


def _gemm(bexp, xs, expert_W_bf16, expert_b):
    grid_spec = pltpu.PrefetchScalarGridSpec(
        num_scalar_prefetch=1,
        grid=(NG,),
        in_specs=[
            pl.BlockSpec((SM, NUM_IN), lambda b, s: (b, 0)),
            pl.BlockSpec((E, NUM_IN, TOTAL_OUT), lambda b, s: (0, 0, 0)),
            pl.BlockSpec((E, TOTAL_OUT), lambda b, s: (0, 0)),
        ],
        out_specs=[
            pl.BlockSpec((SM, HALF_OUT), lambda b, s: (b, 0)),
            pl.BlockSpec((SM, HALF_OUT), lambda b, s: (b, 0)),
        ],
    )
    return pl.pallas_call(
        _gemm_body,
        grid_spec=grid_spec,
        out_shape=[jax.ShapeDtypeStruct((CAP, HALF_OUT), jnp.float32),
                   jax.ShapeDtypeStruct((CAP, HALF_OUT), jnp.float32)],
    )(bexp, xs, expert_W_bf16, expert_b)


def _gath_body(pos_hbm, y1_hbm, y2_hbm, o1_hbm, o2_hbm,
               pv_v, y1_v, y2_v, g1sem, g2sem, wsem):
    wid = _wid()
    pltpu.sync_copy(pos_hbm.at[pl.ds(wid * RPT, RPT)], pv_v)
    g1 = [pltpu.async_copy(y1_hbm.at[pv_v.at[j]],
                           y1_v.at[pl.ds(j * 128, 128)], g1sem)
          for j in range(RPT)]
    g2 = [pltpu.async_copy(y2_hbm.at[pv_v.at[j]],
                           y2_v.at[pl.ds(j * 128, 128)], g2sem)
          for j in range(RPT)]
    for c in g1:
        c.wait()
    w1 = pltpu.async_copy(y1_v, o1_hbm.at[pl.ds(wid * TPT, TPT)], wsem)
    for c in g2:
        c.wait()
    w2 = pltpu.async_copy(y2_v, o2_hbm.at[pl.ds(wid * TPT, TPT)], wsem)
    w1.wait()
    w2.wait()


def _gather_y(pos, y1, y2):
    f = functools.partial(
        pl.kernel,
        out_type=[jax.ShapeDtypeStruct((N_TOK, HALF_OUT), jnp.float32),
                  jax.ShapeDtypeStruct((N_TOK, HALF_OUT), jnp.float32)],
        mesh=_mesh(),
        scratch_types=[
            pltpu.VMEM((RPT, 128), jnp.int32),
            pltpu.VMEM((TPT, HALF_OUT), jnp.float32),
            pltpu.VMEM((TPT, HALF_OUT), jnp.float32),
            pltpu.SemaphoreType.DMA,
            pltpu.SemaphoreType.DMA,
            pltpu.SemaphoreType.DMA,
        ],
        compiler_params=_SC_PARAMS,
    )(_gath_body)
    return f(pos, y1, y2)


def kernel(x, type_embeddings, atom_types, gate_W, gate_b, expert_W, expert_b):
    atom_types = atom_types.astype(jnp.int32)
    em = _expert_map(type_embeddings, gate_W, gate_b)
    pos, bexp, xs = _sort_scatter(em, atom_types, x)
    y1, y2 = _gemm(bexp, xs, expert_W.astype(jnp.bfloat16), expert_b)
    o1, o2 = _gather_y(pos, y1, y2)
    return (o1, o2)
